# Initial kernel scaffold; baseline (speedup 1.0000x reference)
import jax
import jax.numpy as jnp
from jax.experimental import pallas as pl


def kernel(x, edge_index, W1, b1, s1, W2, b2, s2, W3, b3, s3):
    raise NotImplementedError("write your pallas kernel here")


# speedup vs baseline: 7.1404x; 7.1404x over previous
import functools

import jax
import jax.numpy as jnp
from jax import lax
from jax.experimental import pallas as pl
from jax.experimental.pallas import tpu as pltpu
from jax.experimental.pallas import tpu_sc as plsc

N = 10000
N_PAD = 10112
ROWS_PER_TILE = N_PAD // 16
E = 320000
NT = 32
CHUNK = 128
NCHUNK = 80
E_PAD = NT * NCHUNK * CHUNK

_MESH = plsc.VectorSubcoreMesh(core_axis_name="c", subcore_axis_name="s")
_SC_PARAMS = pltpu.CompilerParams(use_tc_tiling_on_sc=False)


def _norm_col(deg_pair):
    deg = deg_pair[0] + deg_pair[1]
    nrm = jnp.where(deg > 0, lax.rsqrt(jnp.maximum(deg, 1.0)), 0.0)
    return nrm[:, 0:1]


def _deg_body(src_hbm, dst_hbm, ones_hbm, zeros_hbm, out_s, out_d,
              idx_s, idx_d, ones_v, acc_s, acc_d):
    c = lax.axis_index("c")
    s = lax.axis_index("s")
    wid = c * 16 + s
    r0 = s * ROWS_PER_TILE
    pltpu.sync_copy(zeros_hbm.at[pl.ds(r0, ROWS_PER_TILE)],
                    acc_s.at[pl.ds(r0, ROWS_PER_TILE)])
    pltpu.sync_copy(zeros_hbm.at[pl.ds(r0, ROWS_PER_TILE)],
                    acc_d.at[pl.ds(r0, ROWS_PER_TILE)])
    pltpu.sync_copy(src_hbm.at[wid], idx_s)
    pltpu.sync_copy(dst_hbm.at[wid], idx_d)
    pltpu.sync_copy(ones_hbm, ones_v)
    plsc.subcore_barrier()

    def body(j, carry):
        pltpu.sync_copy(ones_v, acc_s.at[idx_s.at[j]], add=True)
        pltpu.sync_copy(ones_v, acc_d.at[idx_d.at[j]], add=True)
        return carry

    lax.fori_loop(0, NCHUNK, body, 0)
    plsc.subcore_barrier()
    pltpu.sync_copy(acc_s.at[pl.ds(r0, ROWS_PER_TILE)],
                    out_s.at[c, pl.ds(r0, ROWS_PER_TILE)])
    pltpu.sync_copy(acc_d.at[pl.ds(r0, ROWS_PER_TILE)],
                    out_d.at[c, pl.ds(r0, ROWS_PER_TILE)])


_deg_kernel = pl.kernel(
    _deg_body,
    out_type=(jax.ShapeDtypeStruct((2, N_PAD, 16), jnp.float32),
              jax.ShapeDtypeStruct((2, N_PAD, 16), jnp.float32)),
    mesh=_MESH,
    compiler_params=_SC_PARAMS,
    scratch_types=[
        pltpu.VMEM((NCHUNK, CHUNK), jnp.int32),
        pltpu.VMEM((NCHUNK, CHUNK), jnp.int32),
        pltpu.VMEM((CHUNK, 16), jnp.float32),
        pltpu.VMEM_SHARED((N_PAD, 16), jnp.float32),
        pltpu.VMEM_SHARED((N_PAD, 16), jnp.float32),
    ],
)


def _agg_body(h_hbm, src_hbm, dst_hbm, zeros_hbm, out_hbm,
              idx_s, idx_d, rows, acc, sem):
    c = lax.axis_index("c")
    s = lax.axis_index("s")
    wid = c * 16 + s
    r0 = s * ROWS_PER_TILE
    pltpu.sync_copy(zeros_hbm.at[pl.ds(r0, ROWS_PER_TILE)],
                    acc.at[pl.ds(r0, ROWS_PER_TILE)])
    pltpu.sync_copy(src_hbm.at[wid], idx_s)
    pltpu.sync_copy(dst_hbm.at[wid], idx_d)
    plsc.subcore_barrier()

    def body(j, carry):
        pltpu.async_copy(h_hbm.at[idx_s.at[j]], rows, sem).wait()
        pltpu.sync_copy(rows, acc.at[idx_d.at[j]], add=True)
        return carry

    lax.fori_loop(0, NCHUNK, body, 0)
    plsc.subcore_barrier()
    pltpu.sync_copy(acc.at[pl.ds(r0, ROWS_PER_TILE)],
                    out_hbm.at[c, pl.ds(r0, ROWS_PER_TILE)])


def _make_agg(feat):
    return pl.kernel(
        _agg_body,
        out_type=jax.ShapeDtypeStruct((2, N_PAD, feat), jnp.float32),
        mesh=_MESH,
        compiler_params=_SC_PARAMS,
        scratch_types=[
            pltpu.VMEM((NCHUNK, CHUNK), jnp.int32),
            pltpu.VMEM((NCHUNK, CHUNK), jnp.int32),
            pltpu.VMEM((CHUNK, feat), jnp.float32),
            pltpu.VMEM_SHARED((N_PAD, feat), jnp.float32),
            pltpu.SemaphoreType.DMA,
        ],
    )


_agg64 = _make_agg(64)
_agg32 = _make_agg(32)
_agg16 = _make_agg(16)


def _tc_first_body(x_ref, degs_ref, w_ref, out_ref):
    norm_s = _norm_col(degs_ref[...])
    h = jnp.dot(x_ref[...], w_ref[...], preferred_element_type=jnp.float32)
    out_ref[...] = h * norm_s


def _tc_mid_body(p_ref, degd_ref, degs_ref, b_ref, w_ref, out_ref):
    norm_d = _norm_col(degd_ref[...])
    norm_s = _norm_col(degs_ref[...])
    agg = p_ref[0] + p_ref[1]
    z = jnp.maximum(agg * norm_d + b_ref[...], 0.0)
    out_ref[...] = jnp.dot(z, w_ref[...],
                           preferred_element_type=jnp.float32) * norm_s


def _tc_out_body(p_ref, degd_ref, b_ref, out_ref):
    norm_d = _norm_col(degd_ref[...])
    agg = p_ref[0] + p_ref[1]
    z = jnp.maximum(agg * norm_d + b_ref[...], 0.0)
    m = jnp.max(z, axis=1, keepdims=True)
    e = jnp.exp(z - m)
    out_ref[...] = e / jnp.sum(e, axis=1, keepdims=True)


def _tc_first(x, degs, w):
    return pl.pallas_call(
        _tc_first_body,
        out_shape=jax.ShapeDtypeStruct((N_PAD, w.shape[1]), jnp.float32),
    )(x, degs, w)


def _tc_mid(p, degd, degs, b, w):
    return pl.pallas_call(
        _tc_mid_body,
        out_shape=jax.ShapeDtypeStruct((N_PAD, w.shape[1]), jnp.float32),
    )(p, degd, degs, b, w)


def _tc_out(p, degd, b):
    return pl.pallas_call(
        _tc_out_body,
        out_shape=jax.ShapeDtypeStruct((N_PAD, b.shape[1]), jnp.float32),
    )(p, degd, b)


def kernel(x, edge_index, W1, b1, s1, W2, b2, s2, W3, b3, s3):
    pad = E_PAD - E
    fill = jnp.full((pad,), N, dtype=jnp.int32)
    srcp = jnp.concatenate([edge_index[0], fill]).reshape(NT, NCHUNK, CHUNK)
    dstp = jnp.concatenate([edge_index[1], fill]).reshape(NT, NCHUNK, CHUNK)
    x_pad = jnp.concatenate(
        [x, jnp.zeros((N_PAD - N, x.shape[1]), jnp.float32)], axis=0)

    ones16 = jnp.ones((CHUNK, 16), jnp.float32)
    z16 = jnp.zeros((N_PAD, 16), jnp.float32)
    z32 = jnp.zeros((N_PAD, 32), jnp.float32)
    z64 = jnp.zeros((N_PAD, 64), jnp.float32)

    deg_s, deg_d = _deg_kernel(srcp, dstp, ones16, z16)

    h1 = _tc_first(x_pad, deg_s, W1)
    p1 = _agg64(h1, srcp, dstp, z64)
    h2 = _tc_mid(p1, deg_d, deg_s, (b1 + s1).reshape(1, 64), W2)
    p2 = _agg32(h2, srcp, dstp, z32)
    h3 = _tc_mid(p2, deg_d, deg_s, (b2 + s2).reshape(1, 32), W3)
    p3 = _agg16(h3, srcp, dstp, z16)
    out = _tc_out(p3, deg_d, (b3 + s3).reshape(1, 16))
    return out[:N]


# speedup vs baseline: 8.8060x; 1.2333x over previous
import functools

import jax
import jax.numpy as jnp
from jax import lax
from jax.experimental import pallas as pl
from jax.experimental.pallas import tpu as pltpu
from jax.experimental.pallas import tpu_sc as plsc

N = 10000
N_PAD = 10112
ROWS_PER_TILE = N_PAD // 16
E = 320000
NT = 32
CHUNK = 128
NCHUNK = 80
E_PAD = NT * NCHUNK * CHUNK

_MESH = plsc.VectorSubcoreMesh(core_axis_name="c", subcore_axis_name="s")
_SC_PARAMS = pltpu.CompilerParams(use_tc_tiling_on_sc=False)


def _norm_col(deg_pair):
    deg = deg_pair[0] + deg_pair[1]
    nrm = jnp.where(deg > 0, lax.rsqrt(jnp.maximum(deg, 1.0)), 0.0)
    return nrm[:, 0:1]


def _deg_body(src_hbm, dst_hbm, ones_hbm, zeros_hbm, out_s, out_d,
              idx_s, idx_d, ones_v, acc_s, acc_d):
    c = lax.axis_index("c")
    s = lax.axis_index("s")
    wid = c * 16 + s
    r0 = s * ROWS_PER_TILE
    pltpu.sync_copy(zeros_hbm.at[pl.ds(r0, ROWS_PER_TILE)],
                    acc_s.at[pl.ds(r0, ROWS_PER_TILE)])
    pltpu.sync_copy(zeros_hbm.at[pl.ds(r0, ROWS_PER_TILE)],
                    acc_d.at[pl.ds(r0, ROWS_PER_TILE)])
    pltpu.sync_copy(src_hbm.at[wid], idx_s)
    pltpu.sync_copy(dst_hbm.at[wid], idx_d)
    pltpu.sync_copy(ones_hbm, ones_v)
    plsc.subcore_barrier()

    def body(j, carry):
        pltpu.sync_copy(ones_v, acc_s.at[idx_s.at[j]], add=True)
        pltpu.sync_copy(ones_v, acc_d.at[idx_d.at[j]], add=True)
        return carry

    lax.fori_loop(0, NCHUNK, body, 0)
    plsc.subcore_barrier()
    pltpu.sync_copy(acc_s.at[pl.ds(r0, ROWS_PER_TILE)],
                    out_s.at[c, pl.ds(r0, ROWS_PER_TILE)])
    pltpu.sync_copy(acc_d.at[pl.ds(r0, ROWS_PER_TILE)],
                    out_d.at[c, pl.ds(r0, ROWS_PER_TILE)])


_deg_kernel = pl.kernel(
    _deg_body,
    out_type=(jax.ShapeDtypeStruct((2, N_PAD, 16), jnp.float32),
              jax.ShapeDtypeStruct((2, N_PAD, 16), jnp.float32)),
    mesh=_MESH,
    compiler_params=_SC_PARAMS,
    scratch_types=[
        pltpu.VMEM((NCHUNK, CHUNK), jnp.int32),
        pltpu.VMEM((NCHUNK, CHUNK), jnp.int32),
        pltpu.VMEM((CHUNK, 16), jnp.float32),
        pltpu.VMEM_SHARED((N_PAD, 16), jnp.float32),
        pltpu.VMEM_SHARED((N_PAD, 16), jnp.float32),
    ],
)


K = 4
NG = NCHUNK // K


def _agg_body(h_hbm, src_hbm, dst_hbm, zeros_hbm, out_hbm,
              idx_s, idx_d, rows, acc, gsem, ssem):
    c = lax.axis_index("c")
    s = lax.axis_index("s")
    wid = c * 16 + s
    r0 = s * ROWS_PER_TILE
    pltpu.sync_copy(zeros_hbm.at[pl.ds(r0, ROWS_PER_TILE)],
                    acc.at[pl.ds(r0, ROWS_PER_TILE)])
    pltpu.sync_copy(src_hbm.at[wid], idx_s)
    pltpu.sync_copy(dst_hbm.at[wid], idx_d)
    plsc.subcore_barrier()

    def fire_gathers(g, bank):
        for k in range(K):
            pltpu.async_copy(h_hbm.at[idx_s.at[g * K + k]],
                             rows.at[bank, k], gsem.at[bank, k])

    fire_gathers(0, 0)

    def body(g, carry):
        bank = lax.rem(g, 2)

        @pl.when(g < NG - 1)
        def _():
            fire_gathers(g + 1, 1 - bank)

        for k in range(K):
            pltpu.make_async_copy(h_hbm.at[idx_s.at[g * K + k]],
                                  rows.at[bank, k], gsem.at[bank, k]).wait()
            pltpu.async_copy(rows.at[bank, k], acc.at[idx_d.at[g * K + k]],
                             ssem.at[bank, k], add=True)
        for k in range(K):
            pltpu.make_async_copy(rows.at[bank, k],
                                  acc.at[idx_d.at[g * K + k]],
                                  ssem.at[bank, k]).wait()
        return carry

    lax.fori_loop(0, NG, body, 0)
    plsc.subcore_barrier()
    pltpu.sync_copy(acc.at[pl.ds(r0, ROWS_PER_TILE)],
                    out_hbm.at[c, pl.ds(r0, ROWS_PER_TILE)])


def _make_agg(feat):
    return pl.kernel(
        _agg_body,
        out_type=jax.ShapeDtypeStruct((2, N_PAD, feat), jnp.float32),
        mesh=_MESH,
        compiler_params=_SC_PARAMS,
        scratch_types=[
            pltpu.VMEM((NCHUNK, CHUNK), jnp.int32),
            pltpu.VMEM((NCHUNK, CHUNK), jnp.int32),
            pltpu.VMEM((2, K, CHUNK, feat), jnp.float32),
            pltpu.VMEM_SHARED((N_PAD, feat), jnp.float32),
            pltpu.SemaphoreType.DMA((2, K)),
            pltpu.SemaphoreType.DMA((2, K)),
        ],
    )


_agg64 = _make_agg(64)
_agg32 = _make_agg(32)
_agg16 = _make_agg(16)


def _tc_first_body(x_ref, degs_ref, w_ref, out_ref):
    norm_s = _norm_col(degs_ref[...])
    h = jnp.dot(x_ref[...], w_ref[...], preferred_element_type=jnp.float32)
    out_ref[...] = h * norm_s


def _tc_mid_body(p_ref, degd_ref, degs_ref, b_ref, w_ref, out_ref):
    norm_d = _norm_col(degd_ref[...])
    norm_s = _norm_col(degs_ref[...])
    agg = p_ref[0] + p_ref[1]
    z = jnp.maximum(agg * norm_d + b_ref[...], 0.0)
    out_ref[...] = jnp.dot(z, w_ref[...],
                           preferred_element_type=jnp.float32) * norm_s


def _tc_out_body(p_ref, degd_ref, b_ref, out_ref):
    norm_d = _norm_col(degd_ref[...])
    agg = p_ref[0] + p_ref[1]
    z = jnp.maximum(agg * norm_d + b_ref[...], 0.0)
    m = jnp.max(z, axis=1, keepdims=True)
    e = jnp.exp(z - m)
    out_ref[...] = e / jnp.sum(e, axis=1, keepdims=True)


def _tc_first(x, degs, w):
    return pl.pallas_call(
        _tc_first_body,
        out_shape=jax.ShapeDtypeStruct((N_PAD, w.shape[1]), jnp.float32),
    )(x, degs, w)


def _tc_mid(p, degd, degs, b, w):
    return pl.pallas_call(
        _tc_mid_body,
        out_shape=jax.ShapeDtypeStruct((N_PAD, w.shape[1]), jnp.float32),
    )(p, degd, degs, b, w)


def _tc_out(p, degd, b):
    return pl.pallas_call(
        _tc_out_body,
        out_shape=jax.ShapeDtypeStruct((N_PAD, b.shape[1]), jnp.float32),
    )(p, degd, b)


def kernel(x, edge_index, W1, b1, s1, W2, b2, s2, W3, b3, s3):
    pad = E_PAD - E
    fill_s = jnp.full((pad,), N, dtype=jnp.int32)
    fill_d = N + jnp.arange(pad, dtype=jnp.int32) % (N_PAD - N)
    srcp = jnp.concatenate([edge_index[0], fill_s]).reshape(NT, NCHUNK, CHUNK)
    dstp = jnp.concatenate([edge_index[1], fill_d]).reshape(NT, NCHUNK, CHUNK)
    x_pad = jnp.concatenate(
        [x, jnp.zeros((N_PAD - N, x.shape[1]), jnp.float32)], axis=0)

    ones16 = jnp.ones((CHUNK, 16), jnp.float32)
    z16 = jnp.zeros((N_PAD, 16), jnp.float32)
    z32 = jnp.zeros((N_PAD, 32), jnp.float32)
    z64 = jnp.zeros((N_PAD, 64), jnp.float32)

    deg_s, deg_d = _deg_kernel(srcp, dstp, ones16, z16)

    h1 = _tc_first(x_pad, deg_s, W1)
    p1 = _agg64(h1, srcp, dstp, z64)
    h2 = _tc_mid(p1, deg_d, deg_s, (b1 + s1).reshape(1, 64), W2)
    p2 = _agg32(h2, srcp, dstp, z32)
    h3 = _tc_mid(p2, deg_d, deg_s, (b2 + s2).reshape(1, 32), W3)
    p3 = _agg16(h3, srcp, dstp, z16)
    out = _tc_out(p3, deg_d, (b3 + s3).reshape(1, 16))
    return out[:N]


# speedup vs baseline: 15.4603x; 1.7557x over previous
import jax
import jax.numpy as jnp
from jax import lax
from jax.experimental import pallas as pl
from jax.experimental.pallas import tpu as pltpu
from jax.experimental.pallas import tpu_sc as plsc

N = 10000
N_PAD = 10112
ROWS_PER_TILE = N_PAD // 16
E = 320000
NT = 32
CHUNK = 128
NCHUNK = 80
E_PAD = NT * NCHUNK * CHUNK
K = 4
NG = NCHUNK // K

_MESH = plsc.VectorSubcoreMesh(core_axis_name="c", subcore_axis_name="s")
_SC_PARAMS = pltpu.CompilerParams(use_tc_tiling_on_sc=False)


def _norm_col(deg_pair, col):
    deg = deg_pair[0] + deg_pair[1]
    nrm = jnp.where(deg > 0, lax.rsqrt(jnp.maximum(deg, 1.0)), 0.0)
    return nrm[:, col:col + 1]


def _deg_body(src_hbm, dst_hbm, ones_hbm, zeros_hbm, out_hbm,
              idx_s, idx_d, ones_v, acc):
    c = lax.axis_index("c")
    s = lax.axis_index("s")
    wid = c * 16 + s
    r0 = s * ROWS_PER_TILE

    if True:
        pltpu.sync_copy(zeros_hbm.at[pl.ds(r0, ROWS_PER_TILE)],
                        acc.at[pl.ds(r0, ROWS_PER_TILE)])
        pltpu.sync_copy(src_hbm.at[wid], idx_s)
        pltpu.sync_copy(dst_hbm.at[wid], idx_d)
        pltpu.sync_copy(ones_hbm, ones_v)
        plsc.subcore_barrier()

        def body(j, carry):
            pltpu.sync_copy(ones_v.at[0], acc.at[idx_s.at[j]], add=True)
            pltpu.sync_copy(ones_v.at[1], acc.at[idx_d.at[j]], add=True)
            return carry

        lax.fori_loop(0, NCHUNK, body, 0)
        plsc.subcore_barrier()
        pltpu.sync_copy(acc.at[pl.ds(r0, ROWS_PER_TILE)],
                        out_hbm.at[c, pl.ds(r0, ROWS_PER_TILE)])


_deg_kernel = pl.kernel(
    _deg_body,
    out_type=jax.ShapeDtypeStruct((2, N_PAD, 16), jnp.float32),
    mesh=_MESH,
    compiler_params=_SC_PARAMS,
    scratch_types=[
        pltpu.VMEM((NCHUNK, CHUNK), jnp.int32),
        pltpu.VMEM((NCHUNK, CHUNK), jnp.int32),
        pltpu.VMEM((2, CHUNK, 16), jnp.float32),
        pltpu.VMEM_SHARED((N_PAD, 16), jnp.float32),
    ],
)


def _make_agg_body(feat):
    def _agg_body(h_hbm, src_hbm, dst_hbm, zeros_hbm, out_hbm,
                  idx_s, idx_d, rows, h_sp, acc, gsem, ssem):
        c = lax.axis_index("c")
        s = lax.axis_index("s")
        wid = c * 16 + s
        r0 = s * ROWS_PER_TILE

        if True:
            pltpu.sync_copy(h_hbm.at[pl.ds(r0, ROWS_PER_TILE)],
                            h_sp.at[pl.ds(r0, ROWS_PER_TILE)])
            pltpu.sync_copy(zeros_hbm.at[pl.ds(r0, ROWS_PER_TILE)],
                            acc.at[pl.ds(r0, ROWS_PER_TILE)])
            pltpu.sync_copy(src_hbm.at[wid], idx_s)
            pltpu.sync_copy(dst_hbm.at[wid], idx_d)
            plsc.subcore_barrier()

            def fire_gathers(g, bank):
                for k in range(K):
                    pltpu.async_copy(h_sp.at[idx_s.at[g * K + k]],
                                     rows.at[bank, k], gsem.at[bank, k])

            fire_gathers(0, 0)

            def body(g, carry):
                bank = lax.rem(g, 2)

                @pl.when(g < NG - 1)
                def _():
                    fire_gathers(g + 1, 1 - bank)

                for k in range(K):
                    pltpu.make_async_copy(h_sp.at[idx_s.at[g * K + k]],
                                          rows.at[bank, k],
                                          gsem.at[bank, k]).wait()
                    pltpu.async_copy(rows.at[bank, k],
                                     acc.at[idx_d.at[g * K + k]],
                                     ssem.at[bank, k], add=True)
                for k in range(K):
                    pltpu.make_async_copy(rows.at[bank, k],
                                          acc.at[idx_d.at[g * K + k]],
                                          ssem.at[bank, k]).wait()
                return carry

            lax.fori_loop(0, NG, body, 0)
            plsc.subcore_barrier()
            pltpu.sync_copy(acc.at[pl.ds(r0, ROWS_PER_TILE)],
                            out_hbm.at[c, pl.ds(r0, ROWS_PER_TILE)])

    return _agg_body


def _make_agg(feat):
    return pl.kernel(
        _make_agg_body(feat),
        out_type=jax.ShapeDtypeStruct((2, N_PAD, feat), jnp.float32),
        mesh=_MESH,
        compiler_params=_SC_PARAMS,
        scratch_types=[
            pltpu.VMEM((NCHUNK, CHUNK), jnp.int32),
            pltpu.VMEM((NCHUNK, CHUNK), jnp.int32),
            pltpu.VMEM((2, K, CHUNK, feat), jnp.float32),
            pltpu.VMEM_SHARED((N_PAD, feat), jnp.float32),
            pltpu.VMEM_SHARED((N_PAD, feat), jnp.float32),
            pltpu.SemaphoreType.DMA((2, K)),
            pltpu.SemaphoreType.DMA((2, K)),
        ],
    )


_agg32 = _make_agg(32)
_agg16 = _make_agg(16)


def _tc_first_body(x_ref, deg_ref, w_ref, outa_ref, outb_ref):
    norm_s = _norm_col(deg_ref[...], 0)
    h = jnp.dot(x_ref[...], w_ref[...], preferred_element_type=jnp.float32)
    h = h * norm_s
    outa_ref[...] = h[:, :32]
    outb_ref[...] = h[:, 32:]


def _tc_mid2_body(pa_ref, pb_ref, deg_ref, b_ref, w_ref, out_ref):
    norm_d = _norm_col(deg_ref[...], 8)
    norm_s = _norm_col(deg_ref[...], 0)
    agg = jnp.concatenate([pa_ref[0] + pa_ref[1], pb_ref[0] + pb_ref[1]],
                          axis=1)
    z = jnp.maximum(agg * norm_d + b_ref[...], 0.0)
    out_ref[...] = jnp.dot(z, w_ref[...],
                           preferred_element_type=jnp.float32) * norm_s


def _tc_mid_body(p_ref, deg_ref, b_ref, w_ref, out_ref):
    norm_d = _norm_col(deg_ref[...], 8)
    norm_s = _norm_col(deg_ref[...], 0)
    agg = p_ref[0] + p_ref[1]
    z = jnp.maximum(agg * norm_d + b_ref[...], 0.0)
    out_ref[...] = jnp.dot(z, w_ref[...],
                           preferred_element_type=jnp.float32) * norm_s


def _tc_out_body(p_ref, deg_ref, b_ref, out_ref):
    norm_d = _norm_col(deg_ref[...], 8)
    agg = p_ref[0] + p_ref[1]
    z = jnp.maximum(agg * norm_d + b_ref[...], 0.0)
    m = jnp.max(z, axis=1, keepdims=True)
    e = jnp.exp(z - m)
    out_ref[...] = e / jnp.sum(e, axis=1, keepdims=True)


def _tc_first(x, deg, w):
    return pl.pallas_call(
        _tc_first_body,
        out_shape=(jax.ShapeDtypeStruct((N_PAD, 32), jnp.float32),
                   jax.ShapeDtypeStruct((N_PAD, 32), jnp.float32)),
    )(x, deg, w)


def _tc_mid2(pa, pb, deg, b, w):
    return pl.pallas_call(
        _tc_mid2_body,
        out_shape=jax.ShapeDtypeStruct((N_PAD, w.shape[1]), jnp.float32),
    )(pa, pb, deg, b, w)


def _tc_mid(p, deg, b, w):
    return pl.pallas_call(
        _tc_mid_body,
        out_shape=jax.ShapeDtypeStruct((N_PAD, w.shape[1]), jnp.float32),
    )(p, deg, b, w)


def _tc_out(p, deg, b):
    return pl.pallas_call(
        _tc_out_body,
        out_shape=jax.ShapeDtypeStruct((N_PAD, b.shape[1]), jnp.float32),
    )(p, deg, b)


def kernel(x, edge_index, W1, b1, s1, W2, b2, s2, W3, b3, s3):
    pad = E_PAD - E
    fill_s = jnp.full((pad,), N, dtype=jnp.int32)
    fill_d = N + jnp.arange(pad, dtype=jnp.int32) % (N_PAD - N)
    srcp = jnp.concatenate([edge_index[0], fill_s]).reshape(NT, NCHUNK, CHUNK)
    dstp = jnp.concatenate([edge_index[1], fill_d]).reshape(NT, NCHUNK, CHUNK)
    x_pad = jnp.concatenate(
        [x, jnp.zeros((N_PAD - N, x.shape[1]), jnp.float32)], axis=0)

    eye = jnp.zeros((2, 1, 16), jnp.float32).at[0, 0, 0].set(1.0)
    eye = eye.at[1, 0, 8].set(1.0)
    ones2 = jnp.broadcast_to(eye, (2, CHUNK, 16))
    z16 = jnp.zeros((N_PAD, 16), jnp.float32)
    z32 = jnp.zeros((N_PAD, 32), jnp.float32)

    deg = _deg_kernel(srcp, dstp, ones2, z16)

    h1a, h1b = _tc_first(x_pad, deg, W1)
    p1a = _agg32(h1a, srcp, dstp, z32)
    p1b = _agg32(h1b, srcp, dstp, z32)
    h2 = _tc_mid2(p1a, p1b, deg, (b1 + s1).reshape(1, 64), W2)
    p2 = _agg32(h2, srcp, dstp, z32)
    h3 = _tc_mid(p2, deg, (b2 + s2).reshape(1, 32), W3)
    p3 = _agg16(h3, srcp, dstp, z16)
    out = _tc_out(p3, deg, (b3 + s3).reshape(1, 16))
    return out[:N]


# speedup vs baseline: 15.8291x; 1.0239x over previous
import jax
import jax.numpy as jnp
from jax import lax
from jax.experimental import pallas as pl
from jax.experimental.pallas import tpu as pltpu
from jax.experimental.pallas import tpu_sc as plsc

N = 10000
N_PAD = 10112
ROWS_PER_TILE = N_PAD // 16
E = 320000
NT = 32
CHUNK = 128
NCHUNK = 80
E_PAD = NT * NCHUNK * CHUNK
K = 4
NG = NCHUNK // K

_MESH = plsc.VectorSubcoreMesh(core_axis_name="c", subcore_axis_name="s")
_SC_PARAMS = pltpu.CompilerParams(use_tc_tiling_on_sc=False)


def _norm_col(deg_pair, col):
    deg = deg_pair[0] + deg_pair[1]
    nrm = jnp.where(deg > 0, lax.rsqrt(jnp.maximum(deg, 1.0)), 0.0)
    return nrm[:, col:col + 1]


def _deg_body(src_hbm, dst_hbm, ones_hbm, zeros_hbm, out_hbm,
              idx_s, idx_d, ones_v, acc):
    c = lax.axis_index("c")
    s = lax.axis_index("s")
    wid = c * 16 + s
    r0 = s * ROWS_PER_TILE

    if True:
        pltpu.sync_copy(zeros_hbm.at[pl.ds(r0, ROWS_PER_TILE)],
                        acc.at[pl.ds(r0, ROWS_PER_TILE)])
        pltpu.sync_copy(src_hbm.at[wid], idx_s)
        pltpu.sync_copy(dst_hbm.at[wid], idx_d)
        pltpu.sync_copy(ones_hbm, ones_v)
        plsc.subcore_barrier()

        def body(j, carry):
            pltpu.sync_copy(ones_v.at[0], acc.at[idx_s.at[j]], add=True)
            pltpu.sync_copy(ones_v.at[1], acc.at[idx_d.at[j]], add=True)
            return carry

        lax.fori_loop(0, NCHUNK, body, 0)
        plsc.subcore_barrier()
        pltpu.sync_copy(acc.at[pl.ds(r0, ROWS_PER_TILE)],
                        out_hbm.at[c, pl.ds(r0, ROWS_PER_TILE)])


_deg_kernel = pl.kernel(
    _deg_body,
    out_type=jax.ShapeDtypeStruct((2, N_PAD, 8), jnp.float32),
    mesh=_MESH,
    compiler_params=_SC_PARAMS,
    scratch_types=[
        pltpu.VMEM((NCHUNK, CHUNK), jnp.int32),
        pltpu.VMEM((NCHUNK, CHUNK), jnp.int32),
        pltpu.VMEM((2, CHUNK, 8), jnp.float32),
        pltpu.VMEM_SHARED((N_PAD, 8), jnp.float32),
    ],
)


def _make_agg_body(feat):
    def _agg_body(h_hbm, src_hbm, dst_hbm, zeros_hbm, out_hbm,
                  idx_s, idx_d, rows, h_sp, acc, gsem, ssem):
        c = lax.axis_index("c")
        s = lax.axis_index("s")
        wid = c * 16 + s
        r0 = s * ROWS_PER_TILE

        if True:
            pltpu.sync_copy(h_hbm.at[pl.ds(r0, ROWS_PER_TILE)],
                            h_sp.at[pl.ds(r0, ROWS_PER_TILE)])
            pltpu.sync_copy(zeros_hbm.at[pl.ds(r0, ROWS_PER_TILE)],
                            acc.at[pl.ds(r0, ROWS_PER_TILE)])
            pltpu.sync_copy(src_hbm.at[wid], idx_s)
            pltpu.sync_copy(dst_hbm.at[wid], idx_d)
            plsc.subcore_barrier()

            def fire_gathers(g, bank):
                for k in range(K):
                    pltpu.async_copy(h_sp.at[idx_s.at[g * K + k]],
                                     rows.at[bank, k], gsem.at[bank, k])

            fire_gathers(0, 0)

            def body(g, carry):
                bank = lax.rem(g, 2)

                @pl.when(g < NG - 1)
                def _():
                    fire_gathers(g + 1, 1 - bank)

                for k in range(K):
                    pltpu.make_async_copy(h_sp.at[idx_s.at[g * K + k]],
                                          rows.at[bank, k],
                                          gsem.at[bank, k]).wait()
                    pltpu.async_copy(rows.at[bank, k],
                                     acc.at[idx_d.at[g * K + k]],
                                     ssem.at[bank, k], add=True)
                for k in range(K):
                    pltpu.make_async_copy(rows.at[bank, k],
                                          acc.at[idx_d.at[g * K + k]],
                                          ssem.at[bank, k]).wait()
                return carry

            lax.fori_loop(0, NG, body, 0)
            plsc.subcore_barrier()
            pltpu.sync_copy(acc.at[pl.ds(r0, ROWS_PER_TILE)],
                            out_hbm.at[c, pl.ds(r0, ROWS_PER_TILE)])

    return _agg_body


def _make_agg(feat):
    return pl.kernel(
        _make_agg_body(feat),
        out_type=jax.ShapeDtypeStruct((2, N_PAD, feat), jnp.float32),
        mesh=_MESH,
        compiler_params=_SC_PARAMS,
        scratch_types=[
            pltpu.VMEM((NCHUNK, CHUNK), jnp.int32),
            pltpu.VMEM((NCHUNK, CHUNK), jnp.int32),
            pltpu.VMEM((2, K, CHUNK, feat), jnp.float32),
            pltpu.VMEM_SHARED((N_PAD, feat), jnp.float32),
            pltpu.VMEM_SHARED((N_PAD, feat), jnp.float32),
            pltpu.SemaphoreType.DMA((2, K)),
            pltpu.SemaphoreType.DMA((2, K)),
        ],
    )


_agg32 = _make_agg(32)
_agg16 = _make_agg(16)


_ZPAD = N_PAD - N


def _tc_first_body(x_ref, deg_ref, w_ref, outa_ref, outb_ref):
    norm_s = _norm_col(deg_ref[...], 0)[:N]
    h = jnp.dot(x_ref[...], w_ref[...], preferred_element_type=jnp.float32)
    h = h * norm_s
    zp = jnp.zeros((_ZPAD, 32), jnp.float32)
    outa_ref[...] = jnp.concatenate([h[:, :32], zp], axis=0)
    outb_ref[...] = jnp.concatenate([h[:, 32:], zp], axis=0)


def _tc_mid2_body(pa_ref, pb_ref, deg_ref, b_ref, w_ref, out_ref):
    norm_d = _norm_col(deg_ref[...], 4)
    norm_s = _norm_col(deg_ref[...], 0)
    za = jnp.maximum((pa_ref[0] + pa_ref[1]) * norm_d + b_ref[:, :32], 0.0)
    zb = jnp.maximum((pb_ref[0] + pb_ref[1]) * norm_d + b_ref[:, 32:], 0.0)
    h = (jnp.dot(za, w_ref[:32], preferred_element_type=jnp.float32)
         + jnp.dot(zb, w_ref[32:], preferred_element_type=jnp.float32))
    out_ref[...] = h * norm_s


def _tc_mid_body(p_ref, deg_ref, b_ref, w_ref, out_ref):
    norm_d = _norm_col(deg_ref[...], 4)
    norm_s = _norm_col(deg_ref[...], 0)
    agg = p_ref[0] + p_ref[1]
    z = jnp.maximum(agg * norm_d + b_ref[...], 0.0)
    out_ref[...] = jnp.dot(z, w_ref[...],
                           preferred_element_type=jnp.float32) * norm_s


def _tc_out_body(p_ref, deg_ref, b_ref, out_ref):
    norm_d = _norm_col(deg_ref[...], 4)
    agg = p_ref[0] + p_ref[1]
    z = jnp.maximum(agg * norm_d + b_ref[...], 0.0)
    m = jnp.max(z, axis=1, keepdims=True)
    e = jnp.exp(z - m)
    out_ref[...] = e / jnp.sum(e, axis=1, keepdims=True)


def _tc_first(x, deg, w):
    return pl.pallas_call(
        _tc_first_body,
        out_shape=(jax.ShapeDtypeStruct((N_PAD, 32), jnp.float32),
                   jax.ShapeDtypeStruct((N_PAD, 32), jnp.float32)),
    )(x, deg, w)


def _tc_mid2(pa, pb, deg, b, w):
    return pl.pallas_call(
        _tc_mid2_body,
        out_shape=jax.ShapeDtypeStruct((N_PAD, w.shape[1]), jnp.float32),
    )(pa, pb, deg, b, w)


def _tc_mid(p, deg, b, w):
    return pl.pallas_call(
        _tc_mid_body,
        out_shape=jax.ShapeDtypeStruct((N_PAD, w.shape[1]), jnp.float32),
    )(p, deg, b, w)


def _tc_out(p, deg, b):
    return pl.pallas_call(
        _tc_out_body,
        out_shape=jax.ShapeDtypeStruct((N_PAD, b.shape[1]), jnp.float32),
    )(p, deg, b)


def kernel(x, edge_index, W1, b1, s1, W2, b2, s2, W3, b3, s3):
    pad = E_PAD - E
    fill_s = jnp.full((pad,), N, dtype=jnp.int32)
    fill_d = N + jnp.arange(pad, dtype=jnp.int32) % (N_PAD - N)
    srcp = jnp.concatenate([edge_index[0], fill_s]).reshape(NT, NCHUNK, CHUNK)
    dstp = jnp.concatenate([edge_index[1], fill_d]).reshape(NT, NCHUNK, CHUNK)

    eye = jnp.zeros((2, 1, 8), jnp.float32).at[0, 0, 0].set(1.0)
    eye = eye.at[1, 0, 4].set(1.0)
    ones2 = jnp.broadcast_to(eye, (2, CHUNK, 8))
    z8 = jnp.zeros((N_PAD, 8), jnp.float32)
    z16 = jnp.zeros((N_PAD, 16), jnp.float32)
    z32 = jnp.zeros((N_PAD, 32), jnp.float32)

    deg = _deg_kernel(srcp, dstp, ones2, z8)

    h1a, h1b = _tc_first(x, deg, W1)
    p1a = _agg32(h1a, srcp, dstp, z32)
    p1b = _agg32(h1b, srcp, dstp, z32)
    h2 = _tc_mid2(p1a, p1b, deg, (b1 + s1).reshape(1, 64), W2)
    p2 = _agg32(h2, srcp, dstp, z32)
    h3 = _tc_mid(p2, deg, (b2 + s2).reshape(1, 32), W3)
    p3 = _agg16(h3, srcp, dstp, z16)
    out = _tc_out(p3, deg, (b3 + s3).reshape(1, 16))
    return out[:N]


# speedup vs baseline: 17.1523x; 1.0836x over previous
import jax
import jax.numpy as jnp
from jax import lax
from jax.experimental import pallas as pl
from jax.experimental.pallas import tpu as pltpu
from jax.experimental.pallas import tpu_sc as plsc

N = 10000
N_PAD = 10112
ROWS_PER_TILE = N_PAD // 16
E = 320000
NT = 32
CHUNK = 128
NROWS = E // CHUNK
NCHUNK = NROWS // NT
NEXTRA = NROWS - NCHUNK * NT
K = 3
NG = NCHUNK // K

_MESH = plsc.VectorSubcoreMesh(core_axis_name="c", subcore_axis_name="s")
_SC_PARAMS = pltpu.CompilerParams(use_tc_tiling_on_sc=False)


def _norm_col(deg_pair, col):
    deg = deg_pair[0] + deg_pair[1]
    nrm = jnp.where(deg > 0, lax.rsqrt(jnp.maximum(deg, 1.0)), 0.0)
    return nrm[:, col:col + 1]


def _load_indices(src_hbm, dst_hbm, idx_s, idx_d, wid):
    pltpu.sync_copy(src_hbm.at[pl.ds(NCHUNK * wid, NCHUNK)],
                    idx_s.at[pl.ds(0, NCHUNK)])
    pltpu.sync_copy(dst_hbm.at[pl.ds(NCHUNK * wid, NCHUNK)],
                    idx_d.at[pl.ds(0, NCHUNK)])

    @pl.when(wid < NEXTRA)
    def _():
        pltpu.sync_copy(src_hbm.at[NCHUNK * NT + wid], idx_s.at[NCHUNK])
        pltpu.sync_copy(dst_hbm.at[NCHUNK * NT + wid], idx_d.at[NCHUNK])


def _deg_body(src_hbm, dst_hbm, ones_hbm, zeros_hbm, out_hbm,
              idx_s, idx_d, ones_v, acc):
    c = lax.axis_index("c")
    s = lax.axis_index("s")
    wid = c * 16 + s
    r0 = s * ROWS_PER_TILE
    pltpu.sync_copy(zeros_hbm.at[pl.ds(r0, ROWS_PER_TILE)],
                    acc.at[pl.ds(r0, ROWS_PER_TILE)])
    _load_indices(src_hbm, dst_hbm, idx_s, idx_d, wid)
    pltpu.sync_copy(ones_hbm, ones_v)
    plsc.subcore_barrier()

    def body(j, carry):
        pltpu.sync_copy(ones_v.at[0], acc.at[idx_s.at[j]], add=True)
        pltpu.sync_copy(ones_v.at[1], acc.at[idx_d.at[j]], add=True)
        return carry

    lax.fori_loop(0, NCHUNK, body, 0)

    @pl.when(wid < NEXTRA)
    def _():
        pltpu.sync_copy(ones_v.at[0], acc.at[idx_s.at[NCHUNK]], add=True)
        pltpu.sync_copy(ones_v.at[1], acc.at[idx_d.at[NCHUNK]], add=True)

    plsc.subcore_barrier()
    pltpu.sync_copy(acc.at[pl.ds(r0, ROWS_PER_TILE)],
                    out_hbm.at[c, pl.ds(r0, ROWS_PER_TILE)])


_deg_kernel = pl.kernel(
    _deg_body,
    out_type=jax.ShapeDtypeStruct((2, N_PAD, 8), jnp.float32),
    mesh=_MESH,
    compiler_params=_SC_PARAMS,
    scratch_types=[
        pltpu.VMEM((NCHUNK + 1, CHUNK), jnp.int32),
        pltpu.VMEM((NCHUNK + 1, CHUNK), jnp.int32),
        pltpu.VMEM((2, CHUNK, 8), jnp.float32),
        pltpu.VMEM_SHARED((N_PAD, 8), jnp.float32),
    ],
)


def _make_agg_body(feat):
    def _agg_body(h_hbm, src_hbm, dst_hbm, zeros_hbm, out_hbm,
                  idx_s, idx_d, rows, h_sp, acc, gsem, ssem):
        c = lax.axis_index("c")
        s = lax.axis_index("s")
        wid = c * 16 + s
        r0 = s * ROWS_PER_TILE
        pltpu.sync_copy(h_hbm.at[pl.ds(r0, ROWS_PER_TILE)],
                        h_sp.at[pl.ds(r0, ROWS_PER_TILE)])
        pltpu.sync_copy(zeros_hbm.at[pl.ds(r0, ROWS_PER_TILE)],
                        acc.at[pl.ds(r0, ROWS_PER_TILE)])
        _load_indices(src_hbm, dst_hbm, idx_s, idx_d, wid)
        plsc.subcore_barrier()

        def fire_gathers(g, bank):
            for k in range(K):
                pltpu.async_copy(h_sp.at[idx_s.at[g * K + k]],
                                 rows.at[bank, k], gsem.at[bank, k])

        fire_gathers(0, 0)

        def body(g, carry):
            bank = lax.rem(g, 2)

            @pl.when(g < NG - 1)
            def _():
                fire_gathers(g + 1, 1 - bank)

            for k in range(K):
                pltpu.make_async_copy(h_sp.at[idx_s.at[g * K + k]],
                                      rows.at[bank, k],
                                      gsem.at[bank, k]).wait()
                pltpu.async_copy(rows.at[bank, k],
                                 acc.at[idx_d.at[g * K + k]],
                                 ssem.at[bank, k], add=True)
            for k in range(K):
                pltpu.make_async_copy(rows.at[bank, k],
                                      acc.at[idx_d.at[g * K + k]],
                                      ssem.at[bank, k]).wait()
            return carry

        lax.fori_loop(0, NG, body, 0)

        @pl.when(wid < NEXTRA)
        def _():
            pltpu.sync_copy(h_sp.at[idx_s.at[NCHUNK]], rows.at[0, 0])
            pltpu.sync_copy(rows.at[0, 0], acc.at[idx_d.at[NCHUNK]],
                            add=True)

        plsc.subcore_barrier()
        pltpu.sync_copy(acc.at[pl.ds(r0, ROWS_PER_TILE)],
                        out_hbm.at[c, pl.ds(r0, ROWS_PER_TILE)])

    return _agg_body


def _make_agg(feat):
    return pl.kernel(
        _make_agg_body(feat),
        out_type=jax.ShapeDtypeStruct((2, N_PAD, feat), jnp.float32),
        mesh=_MESH,
        compiler_params=_SC_PARAMS,
        scratch_types=[
            pltpu.VMEM((NCHUNK + 1, CHUNK), jnp.int32),
            pltpu.VMEM((NCHUNK + 1, CHUNK), jnp.int32),
            pltpu.VMEM((2, K, CHUNK, feat), jnp.float32),
            pltpu.VMEM_SHARED((N_PAD, feat), jnp.float32),
            pltpu.VMEM_SHARED((N_PAD, feat), jnp.float32),
            pltpu.SemaphoreType.DMA((2, K)),
            pltpu.SemaphoreType.DMA((2, K)),
        ],
    )


_agg32 = _make_agg(32)
_agg16 = _make_agg(16)


_ZPAD = N_PAD - N


def _tc_first_body(x_ref, deg_ref, w_ref, outa_ref, outb_ref):
    norm_s = _norm_col(deg_ref[...], 0)[:N]
    h = jnp.dot(x_ref[...], w_ref[...], preferred_element_type=jnp.float32)
    h = h * norm_s
    zp = jnp.zeros((_ZPAD, 32), jnp.float32)
    outa_ref[...] = jnp.concatenate([h[:, :32], zp], axis=0)
    outb_ref[...] = jnp.concatenate([h[:, 32:], zp], axis=0)


def _tc_mid2_body(pa_ref, pb_ref, deg_ref, b_ref, w_ref, out_ref):
    norm_d = _norm_col(deg_ref[...], 4)
    norm_s = _norm_col(deg_ref[...], 0)
    za = jnp.maximum((pa_ref[0] + pa_ref[1]) * norm_d + b_ref[:, :32], 0.0)
    zb = jnp.maximum((pb_ref[0] + pb_ref[1]) * norm_d + b_ref[:, 32:], 0.0)
    h = (jnp.dot(za, w_ref[:32], preferred_element_type=jnp.float32)
         + jnp.dot(zb, w_ref[32:], preferred_element_type=jnp.float32))
    out_ref[...] = h * norm_s


def _tc_mid_body(p_ref, deg_ref, b_ref, w_ref, out_ref):
    norm_d = _norm_col(deg_ref[...], 4)
    norm_s = _norm_col(deg_ref[...], 0)
    agg = p_ref[0] + p_ref[1]
    z = jnp.maximum(agg * norm_d + b_ref[...], 0.0)
    out_ref[...] = jnp.dot(z, w_ref[...],
                           preferred_element_type=jnp.float32) * norm_s


def _tc_out_body(p_ref, deg_ref, b_ref, out_ref):
    norm_d = _norm_col(deg_ref[...], 4)
    agg = p_ref[0] + p_ref[1]
    z = jnp.maximum(agg * norm_d + b_ref[...], 0.0)
    m = jnp.max(z, axis=1, keepdims=True)
    e = jnp.exp(z - m)
    out_ref[...] = e / jnp.sum(e, axis=1, keepdims=True)


def _tc_first(x, deg, w):
    return pl.pallas_call(
        _tc_first_body,
        out_shape=(jax.ShapeDtypeStruct((N_PAD, 32), jnp.float32),
                   jax.ShapeDtypeStruct((N_PAD, 32), jnp.float32)),
    )(x, deg, w)


def _tc_mid2(pa, pb, deg, b, w):
    return pl.pallas_call(
        _tc_mid2_body,
        out_shape=jax.ShapeDtypeStruct((N_PAD, w.shape[1]), jnp.float32),
    )(pa, pb, deg, b, w)


def _tc_mid(p, deg, b, w):
    return pl.pallas_call(
        _tc_mid_body,
        out_shape=jax.ShapeDtypeStruct((N_PAD, w.shape[1]), jnp.float32),
    )(p, deg, b, w)


def _tc_out(p, deg, b):
    return pl.pallas_call(
        _tc_out_body,
        out_shape=jax.ShapeDtypeStruct((N_PAD, b.shape[1]), jnp.float32),
    )(p, deg, b)


def kernel(x, edge_index, W1, b1, s1, W2, b2, s2, W3, b3, s3):
    srcv = edge_index[0].reshape(NROWS, CHUNK)
    dstv = edge_index[1].reshape(NROWS, CHUNK)

    eye = jnp.zeros((2, 1, 8), jnp.float32).at[0, 0, 0].set(1.0)
    eye = eye.at[1, 0, 4].set(1.0)
    ones2 = jnp.broadcast_to(eye, (2, CHUNK, 8))
    z8 = jnp.zeros((N_PAD, 8), jnp.float32)
    z16 = jnp.zeros((N_PAD, 16), jnp.float32)
    z32 = jnp.zeros((N_PAD, 32), jnp.float32)

    deg = _deg_kernel(srcv, dstv, ones2, z8)

    h1a, h1b = _tc_first(x, deg, W1)
    p1a = _agg32(h1a, srcv, dstv, z32)
    p1b = _agg32(h1b, srcv, dstv, z32)
    h2 = _tc_mid2(p1a, p1b, deg, (b1 + s1).reshape(1, 64), W2)
    p2 = _agg32(h2, srcv, dstv, z32)
    h3 = _tc_mid(p2, deg, (b2 + s2).reshape(1, 32), W3)
    p3 = _agg16(h3, srcv, dstv, z16)
    out = _tc_out(p3, deg, (b3 + s3).reshape(1, 16))
    return out[:N]


# speedup vs baseline: 17.2490x; 1.0056x over previous
import jax
import jax.numpy as jnp
from jax import lax
from jax.experimental import pallas as pl
from jax.experimental.pallas import tpu as pltpu
from jax.experimental.pallas import tpu_sc as plsc

N = 10000
N_PAD = 10112
ROWS_PER_TILE = N_PAD // 16
E = 320000
NT = 32
CHUNK = 128
NROWS = E // CHUNK
NCHUNK = NROWS // NT
NEXTRA = NROWS - NCHUNK * NT
K = 3
NG = NCHUNK // K

_MESH = plsc.VectorSubcoreMesh(core_axis_name="c", subcore_axis_name="s")
_SC_PARAMS = pltpu.CompilerParams(use_tc_tiling_on_sc=False)


def _norm_col(deg_pair, col):
    deg = deg_pair[0] + deg_pair[1]
    nrm = jnp.where(deg > 0, lax.rsqrt(jnp.maximum(deg, 1.0)), 0.0)
    return nrm[:, col:col + 1]


def _load_indices(src_hbm, dst_hbm, idx_s, idx_d, wid):
    pltpu.sync_copy(src_hbm.at[pl.ds(NCHUNK * wid, NCHUNK)],
                    idx_s.at[pl.ds(0, NCHUNK)])
    pltpu.sync_copy(dst_hbm.at[pl.ds(NCHUNK * wid, NCHUNK)],
                    idx_d.at[pl.ds(0, NCHUNK)])

    @pl.when(wid < NEXTRA)
    def _():
        pltpu.sync_copy(src_hbm.at[NCHUNK * NT + wid], idx_s.at[NCHUNK])
        pltpu.sync_copy(dst_hbm.at[NCHUNK * NT + wid], idx_d.at[NCHUNK])


def _deg_body(src_hbm, dst_hbm, ones_hbm, zeros_hbm, out_hbm,
              idx_s, idx_d, ones_v, acc):
    c = lax.axis_index("c")
    s = lax.axis_index("s")
    wid = c * 16 + s
    r0 = s * ROWS_PER_TILE
    pltpu.sync_copy(zeros_hbm.at[pl.ds(r0, ROWS_PER_TILE)],
                    acc.at[pl.ds(r0, ROWS_PER_TILE)])
    _load_indices(src_hbm, dst_hbm, idx_s, idx_d, wid)
    pltpu.sync_copy(ones_hbm, ones_v)
    plsc.subcore_barrier()

    def body(j, carry):
        pltpu.sync_copy(ones_v.at[0], acc.at[idx_s.at[j]], add=True)
        pltpu.sync_copy(ones_v.at[1], acc.at[idx_d.at[j]], add=True)
        return carry

    lax.fori_loop(0, NCHUNK, body, 0)

    @pl.when(wid < NEXTRA)
    def _():
        pltpu.sync_copy(ones_v.at[0], acc.at[idx_s.at[NCHUNK]], add=True)
        pltpu.sync_copy(ones_v.at[1], acc.at[idx_d.at[NCHUNK]], add=True)

    plsc.subcore_barrier()
    pltpu.sync_copy(acc.at[pl.ds(r0, ROWS_PER_TILE)],
                    out_hbm.at[c, pl.ds(r0, ROWS_PER_TILE)])


_deg_kernel = pl.kernel(
    _deg_body,
    out_type=jax.ShapeDtypeStruct((2, N_PAD, 8), jnp.float32),
    mesh=_MESH,
    compiler_params=_SC_PARAMS,
    scratch_types=[
        pltpu.VMEM((NCHUNK + 1, CHUNK), jnp.int32),
        pltpu.VMEM((NCHUNK + 1, CHUNK), jnp.int32),
        pltpu.VMEM((2, CHUNK, 8), jnp.float32),
        pltpu.VMEM_SHARED((N_PAD, 8), jnp.float32),
    ],
)


def _agg_pass(h_sp, acc, idx_s, idx_d, rows, gsem, ssem, wid):
    def fire_gathers(g, bank):
        for k in range(K):
            pltpu.async_copy(h_sp.at[idx_s.at[g * K + k]],
                             rows.at[bank, k], gsem.at[bank, k])

    fire_gathers(0, 0)

    def body(g, carry):
        bank = lax.rem(g, 2)

        @pl.when(g < NG - 1)
        def _():
            fire_gathers(g + 1, 1 - bank)

        for k in range(K):
            pltpu.make_async_copy(h_sp.at[idx_s.at[g * K + k]],
                                  rows.at[bank, k],
                                  gsem.at[bank, k]).wait()
            pltpu.async_copy(rows.at[bank, k],
                             acc.at[idx_d.at[g * K + k]],
                             ssem.at[bank, k], add=True)
        for k in range(K):
            pltpu.make_async_copy(rows.at[bank, k],
                                  acc.at[idx_d.at[g * K + k]],
                                  ssem.at[bank, k]).wait()
        return carry

    lax.fori_loop(0, NG, body, 0)

    @pl.when(wid < NEXTRA)
    def _():
        pltpu.sync_copy(h_sp.at[idx_s.at[NCHUNK]], rows.at[0, 0])
        pltpu.sync_copy(rows.at[0, 0], acc.at[idx_d.at[NCHUNK]], add=True)


def _make_agg_body(feat):
    def _agg_body(h_hbm, src_hbm, dst_hbm, zeros_hbm, out_hbm,
                  idx_s, idx_d, rows, h_sp, acc, gsem, ssem):
        c = lax.axis_index("c")
        s = lax.axis_index("s")
        wid = c * 16 + s
        r0 = s * ROWS_PER_TILE
        pltpu.sync_copy(h_hbm.at[pl.ds(r0, ROWS_PER_TILE)],
                        h_sp.at[pl.ds(r0, ROWS_PER_TILE)])
        pltpu.sync_copy(zeros_hbm.at[pl.ds(r0, ROWS_PER_TILE)],
                        acc.at[pl.ds(r0, ROWS_PER_TILE)])
        _load_indices(src_hbm, dst_hbm, idx_s, idx_d, wid)
        plsc.subcore_barrier()
        _agg_pass(h_sp, acc, idx_s, idx_d, rows, gsem, ssem, wid)
        plsc.subcore_barrier()
        pltpu.sync_copy(acc.at[pl.ds(r0, ROWS_PER_TILE)],
                        out_hbm.at[c, pl.ds(r0, ROWS_PER_TILE)])

    return _agg_body


def _l1_body(ha_hbm, hb_hbm, src_hbm, dst_hbm, zeros_hbm, outa_hbm, outb_hbm,
             idx_s, idx_d, rows, h_spa, h_spb, acc, gsem, ssem):
    c = lax.axis_index("c")
    s = lax.axis_index("s")
    wid = c * 16 + s
    r0 = s * ROWS_PER_TILE
    pltpu.sync_copy(ha_hbm.at[pl.ds(r0, ROWS_PER_TILE)],
                    h_spa.at[pl.ds(r0, ROWS_PER_TILE)])
    pltpu.sync_copy(hb_hbm.at[pl.ds(r0, ROWS_PER_TILE)],
                    h_spb.at[pl.ds(r0, ROWS_PER_TILE)])
    pltpu.sync_copy(zeros_hbm.at[pl.ds(r0, ROWS_PER_TILE)],
                    acc.at[pl.ds(r0, ROWS_PER_TILE)])
    _load_indices(src_hbm, dst_hbm, idx_s, idx_d, wid)
    plsc.subcore_barrier()
    _agg_pass(h_spa, acc, idx_s, idx_d, rows, gsem, ssem, wid)
    plsc.subcore_barrier()
    pltpu.sync_copy(acc.at[pl.ds(r0, ROWS_PER_TILE)],
                    outa_hbm.at[c, pl.ds(r0, ROWS_PER_TILE)])
    pltpu.sync_copy(zeros_hbm.at[pl.ds(r0, ROWS_PER_TILE)],
                    acc.at[pl.ds(r0, ROWS_PER_TILE)])
    plsc.subcore_barrier()
    _agg_pass(h_spb, acc, idx_s, idx_d, rows, gsem, ssem, wid)
    plsc.subcore_barrier()
    pltpu.sync_copy(acc.at[pl.ds(r0, ROWS_PER_TILE)],
                    outb_hbm.at[c, pl.ds(r0, ROWS_PER_TILE)])


_l1_kernel = pl.kernel(
    _l1_body,
    out_type=(jax.ShapeDtypeStruct((2, N_PAD, 32), jnp.float32),
              jax.ShapeDtypeStruct((2, N_PAD, 32), jnp.float32)),
    mesh=_MESH,
    compiler_params=_SC_PARAMS,
    scratch_types=[
        pltpu.VMEM((NCHUNK + 1, CHUNK), jnp.int32),
        pltpu.VMEM((NCHUNK + 1, CHUNK), jnp.int32),
        pltpu.VMEM((2, K, CHUNK, 32), jnp.float32),
        pltpu.VMEM_SHARED((N_PAD, 32), jnp.float32),
        pltpu.VMEM_SHARED((N_PAD, 32), jnp.float32),
        pltpu.VMEM_SHARED((N_PAD, 32), jnp.float32),
        pltpu.SemaphoreType.DMA((2, K)),
        pltpu.SemaphoreType.DMA((2, K)),
    ],
)


def _make_agg(feat):
    return pl.kernel(
        _make_agg_body(feat),
        out_type=jax.ShapeDtypeStruct((2, N_PAD, feat), jnp.float32),
        mesh=_MESH,
        compiler_params=_SC_PARAMS,
        scratch_types=[
            pltpu.VMEM((NCHUNK + 1, CHUNK), jnp.int32),
            pltpu.VMEM((NCHUNK + 1, CHUNK), jnp.int32),
            pltpu.VMEM((2, K, CHUNK, feat), jnp.float32),
            pltpu.VMEM_SHARED((N_PAD, feat), jnp.float32),
            pltpu.VMEM_SHARED((N_PAD, feat), jnp.float32),
            pltpu.SemaphoreType.DMA((2, K)),
            pltpu.SemaphoreType.DMA((2, K)),
        ],
    )


_agg32 = _make_agg(32)
_agg16 = _make_agg(16)


_ZPAD = N_PAD - N


def _tc_first_body(x_ref, deg_ref, w_ref, outa_ref, outb_ref):
    norm_s = _norm_col(deg_ref[...], 0)[:N]
    h = jnp.dot(x_ref[...], w_ref[...], preferred_element_type=jnp.float32)
    h = h * norm_s
    zp = jnp.zeros((_ZPAD, 32), jnp.float32)
    outa_ref[...] = jnp.concatenate([h[:, :32], zp], axis=0)
    outb_ref[...] = jnp.concatenate([h[:, 32:], zp], axis=0)


def _tc_mid2_body(pa_ref, pb_ref, deg_ref, b_ref, w_ref, out_ref):
    norm_d = _norm_col(deg_ref[...], 4)
    norm_s = _norm_col(deg_ref[...], 0)
    za = jnp.maximum((pa_ref[0] + pa_ref[1]) * norm_d + b_ref[:, :32], 0.0)
    zb = jnp.maximum((pb_ref[0] + pb_ref[1]) * norm_d + b_ref[:, 32:], 0.0)
    h = (jnp.dot(za, w_ref[:32], preferred_element_type=jnp.float32)
         + jnp.dot(zb, w_ref[32:], preferred_element_type=jnp.float32))
    out_ref[...] = h * norm_s


def _tc_mid_body(p_ref, deg_ref, b_ref, w_ref, out_ref):
    norm_d = _norm_col(deg_ref[...], 4)
    norm_s = _norm_col(deg_ref[...], 0)
    agg = p_ref[0] + p_ref[1]
    z = jnp.maximum(agg * norm_d + b_ref[...], 0.0)
    out_ref[...] = jnp.dot(z, w_ref[...],
                           preferred_element_type=jnp.float32) * norm_s


def _tc_out_body(p_ref, deg_ref, b_ref, out_ref):
    norm_d = _norm_col(deg_ref[...], 4)
    agg = p_ref[0] + p_ref[1]
    z = jnp.maximum(agg * norm_d + b_ref[...], 0.0)
    m = jnp.max(z, axis=1, keepdims=True)
    e = jnp.exp(z - m)
    out_ref[...] = e / jnp.sum(e, axis=1, keepdims=True)


def _tc_first(x, deg, w):
    return pl.pallas_call(
        _tc_first_body,
        out_shape=(jax.ShapeDtypeStruct((N_PAD, 32), jnp.float32),
                   jax.ShapeDtypeStruct((N_PAD, 32), jnp.float32)),
    )(x, deg, w)


def _tc_mid2(pa, pb, deg, b, w):
    return pl.pallas_call(
        _tc_mid2_body,
        out_shape=jax.ShapeDtypeStruct((N_PAD, w.shape[1]), jnp.float32),
    )(pa, pb, deg, b, w)


def _tc_mid(p, deg, b, w):
    return pl.pallas_call(
        _tc_mid_body,
        out_shape=jax.ShapeDtypeStruct((N_PAD, w.shape[1]), jnp.float32),
    )(p, deg, b, w)


def _tc_out(p, deg, b):
    return pl.pallas_call(
        _tc_out_body,
        out_shape=jax.ShapeDtypeStruct((N_PAD, b.shape[1]), jnp.float32),
    )(p, deg, b)


def kernel(x, edge_index, W1, b1, s1, W2, b2, s2, W3, b3, s3):
    srcv = edge_index[0].reshape(NROWS, CHUNK)
    dstv = edge_index[1].reshape(NROWS, CHUNK)

    eye = jnp.zeros((2, 1, 8), jnp.float32).at[0, 0, 0].set(1.0)
    eye = eye.at[1, 0, 4].set(1.0)
    ones2 = jnp.broadcast_to(eye, (2, CHUNK, 8))
    z8 = jnp.zeros((N_PAD, 8), jnp.float32)
    z16 = jnp.zeros((N_PAD, 16), jnp.float32)
    z32 = jnp.zeros((N_PAD, 32), jnp.float32)

    deg = _deg_kernel(srcv, dstv, ones2, z8)

    h1a, h1b = _tc_first(x, deg, W1)
    p1a, p1b = _l1_kernel(h1a, h1b, srcv, dstv, z32)
    h2 = _tc_mid2(p1a, p1b, deg, (b1 + s1).reshape(1, 64), W2)
    p2 = _agg32(h2, srcv, dstv, z32)
    h3 = _tc_mid(p2, deg, (b2 + s2).reshape(1, 32), W3)
    p3 = _agg16(h3, srcv, dstv, z16)
    out = _tc_out(p3, deg, (b3 + s3).reshape(1, 16))
    return out[:N]


# speedup vs baseline: 18.3206x; 1.0621x over previous
import jax
import jax.numpy as jnp
from jax import lax
from jax.experimental import pallas as pl
from jax.experimental.pallas import tpu as pltpu
from jax.experimental.pallas import tpu_sc as plsc

N = 10000
N_PAD = 10112
ROWS_PER_TILE = N_PAD // 16
E = 320000
NT = 32
CHUNK = 128
NROWS = E // CHUNK
NCHUNK = NROWS // NT
NEXTRA = NROWS - NCHUNK * NT
K = 6
NG = NCHUNK // K

_MESH = plsc.VectorSubcoreMesh(core_axis_name="c", subcore_axis_name="s")
_SC_PARAMS = pltpu.CompilerParams(use_tc_tiling_on_sc=False)


def _norm_col(deg_pair, col):
    deg = deg_pair[0] + deg_pair[1]
    nrm = jnp.where(deg > 0, lax.rsqrt(jnp.maximum(deg, 1.0)), 0.0)
    return nrm[:, col:col + 1]


def _load_indices(ei_hbm, idx_s, idx_d, wid):
    pltpu.sync_copy(ei_hbm.at[pl.ds(NCHUNK * wid, NCHUNK), 0],
                    idx_s.at[pl.ds(0, NCHUNK)])
    pltpu.sync_copy(ei_hbm.at[pl.ds(NCHUNK * wid, NCHUNK), 1],
                    idx_d.at[pl.ds(0, NCHUNK)])

    @pl.when(wid < NEXTRA)
    def _():
        pltpu.sync_copy(ei_hbm.at[NCHUNK * NT + wid, 0], idx_s.at[NCHUNK])
        pltpu.sync_copy(ei_hbm.at[NCHUNK * NT + wid, 1], idx_d.at[NCHUNK])


def _deg_body(ei_hbm, ones_hbm, zeros_hbm, out_hbm,
              idx_s, idx_d, ones_v, acc):
    c = lax.axis_index("c")
    s = lax.axis_index("s")
    wid = c * 16 + s
    r0 = s * ROWS_PER_TILE
    pltpu.sync_copy(zeros_hbm.at[pl.ds(r0, ROWS_PER_TILE)],
                    acc.at[pl.ds(r0, ROWS_PER_TILE)])
    _load_indices(ei_hbm, idx_s, idx_d, wid)
    pltpu.sync_copy(ones_hbm, ones_v)
    plsc.subcore_barrier()

    def body(j, carry):
        pltpu.sync_copy(ones_v.at[0], acc.at[idx_s.at[j]], add=True)
        pltpu.sync_copy(ones_v.at[1], acc.at[idx_d.at[j]], add=True)
        return carry

    lax.fori_loop(0, NCHUNK, body, 0)

    @pl.when(wid < NEXTRA)
    def _():
        pltpu.sync_copy(ones_v.at[0], acc.at[idx_s.at[NCHUNK]], add=True)
        pltpu.sync_copy(ones_v.at[1], acc.at[idx_d.at[NCHUNK]], add=True)

    plsc.subcore_barrier()
    pltpu.sync_copy(acc.at[pl.ds(r0, ROWS_PER_TILE)],
                    out_hbm.at[c, pl.ds(r0, ROWS_PER_TILE)])


_deg_kernel = pl.kernel(
    _deg_body,
    out_type=jax.ShapeDtypeStruct((2, N_PAD, 8), jnp.float32),
    mesh=_MESH,
    compiler_params=_SC_PARAMS,
    scratch_types=[
        pltpu.VMEM((NCHUNK + 1, CHUNK), jnp.int32),
        pltpu.VMEM((NCHUNK + 1, CHUNK), jnp.int32),
        pltpu.VMEM((2, CHUNK, 8), jnp.float32),
        pltpu.VMEM_SHARED((N_PAD, 8), jnp.float32),
    ],
)


def _agg_pass(h_sp, acc, idx_s, idx_d, rows, gsem, ssem, wid):
    def fire_gathers(g, bank):
        for k in range(K):
            pltpu.async_copy(h_sp.at[idx_s.at[g * K + k]],
                             rows.at[bank, k], gsem.at[bank, k])

    fire_gathers(0, 0)

    def body(g, carry):
        bank = lax.rem(g, 2)

        @pl.when(g < NG - 1)
        def _():
            fire_gathers(g + 1, 1 - bank)

        for k in range(K):
            pltpu.make_async_copy(h_sp.at[idx_s.at[g * K + k]],
                                  rows.at[bank, k],
                                  gsem.at[bank, k]).wait()
            pltpu.async_copy(rows.at[bank, k],
                             acc.at[idx_d.at[g * K + k]],
                             ssem.at[bank, k], add=True)
        for k in range(K):
            pltpu.make_async_copy(rows.at[bank, k],
                                  acc.at[idx_d.at[g * K + k]],
                                  ssem.at[bank, k]).wait()
        return carry

    lax.fori_loop(0, NG, body, 0)

    @pl.when(wid < NEXTRA)
    def _():
        pltpu.sync_copy(h_sp.at[idx_s.at[NCHUNK]], rows.at[0, 0])
        pltpu.sync_copy(rows.at[0, 0], acc.at[idx_d.at[NCHUNK]], add=True)


def _make_agg_body(feat):
    def _agg_body(h_hbm, ei_hbm, zeros_hbm, out_hbm,
                  idx_s, idx_d, rows, h_sp, acc, gsem, ssem):
        c = lax.axis_index("c")
        s = lax.axis_index("s")
        wid = c * 16 + s
        r0 = s * ROWS_PER_TILE
        pltpu.sync_copy(h_hbm.at[pl.ds(r0, ROWS_PER_TILE)],
                        h_sp.at[pl.ds(r0, ROWS_PER_TILE)])
        pltpu.sync_copy(zeros_hbm.at[pl.ds(r0, ROWS_PER_TILE)],
                        acc.at[pl.ds(r0, ROWS_PER_TILE)])
        _load_indices(ei_hbm, idx_s, idx_d, wid)
        plsc.subcore_barrier()
        _agg_pass(h_sp, acc, idx_s, idx_d, rows, gsem, ssem, wid)
        plsc.subcore_barrier()
        pltpu.sync_copy(acc.at[pl.ds(r0, ROWS_PER_TILE)],
                        out_hbm.at[c, pl.ds(r0, ROWS_PER_TILE)])

    return _agg_body


def _l1_body(ha_hbm, hb_hbm, ei_hbm, zeros_hbm, outa_hbm, outb_hbm,
             idx_s, idx_d, rows, h_spa, h_spb, acc, gsem, ssem):
    c = lax.axis_index("c")
    s = lax.axis_index("s")
    wid = c * 16 + s
    r0 = s * ROWS_PER_TILE
    pltpu.sync_copy(ha_hbm.at[pl.ds(r0, ROWS_PER_TILE)],
                    h_spa.at[pl.ds(r0, ROWS_PER_TILE)])
    pltpu.sync_copy(hb_hbm.at[pl.ds(r0, ROWS_PER_TILE)],
                    h_spb.at[pl.ds(r0, ROWS_PER_TILE)])
    pltpu.sync_copy(zeros_hbm.at[pl.ds(r0, ROWS_PER_TILE)],
                    acc.at[pl.ds(r0, ROWS_PER_TILE)])
    _load_indices(ei_hbm, idx_s, idx_d, wid)
    plsc.subcore_barrier()
    _agg_pass(h_spa, acc, idx_s, idx_d, rows, gsem, ssem, wid)
    plsc.subcore_barrier()
    pltpu.sync_copy(acc.at[pl.ds(r0, ROWS_PER_TILE)],
                    outa_hbm.at[c, pl.ds(r0, ROWS_PER_TILE)])
    pltpu.sync_copy(zeros_hbm.at[pl.ds(r0, ROWS_PER_TILE)],
                    acc.at[pl.ds(r0, ROWS_PER_TILE)])
    plsc.subcore_barrier()
    _agg_pass(h_spb, acc, idx_s, idx_d, rows, gsem, ssem, wid)
    plsc.subcore_barrier()
    pltpu.sync_copy(acc.at[pl.ds(r0, ROWS_PER_TILE)],
                    outb_hbm.at[c, pl.ds(r0, ROWS_PER_TILE)])


_l1_kernel = pl.kernel(
    _l1_body,
    out_type=(jax.ShapeDtypeStruct((2, N_PAD, 32), jnp.float32),
              jax.ShapeDtypeStruct((2, N_PAD, 32), jnp.float32)),
    mesh=_MESH,
    compiler_params=_SC_PARAMS,
    scratch_types=[
        pltpu.VMEM((NCHUNK + 1, CHUNK), jnp.int32),
        pltpu.VMEM((NCHUNK + 1, CHUNK), jnp.int32),
        pltpu.VMEM((2, K, CHUNK, 32), jnp.float32),
        pltpu.VMEM_SHARED((N_PAD, 32), jnp.float32),
        pltpu.VMEM_SHARED((N_PAD, 32), jnp.float32),
        pltpu.VMEM_SHARED((N_PAD, 32), jnp.float32),
        pltpu.SemaphoreType.DMA((2, K)),
        pltpu.SemaphoreType.DMA((2, K)),
    ],
)


def _make_agg(feat):
    return pl.kernel(
        _make_agg_body(feat),
        out_type=jax.ShapeDtypeStruct((2, N_PAD, feat), jnp.float32),
        mesh=_MESH,
        compiler_params=_SC_PARAMS,
        scratch_types=[
            pltpu.VMEM((NCHUNK + 1, CHUNK), jnp.int32),
            pltpu.VMEM((NCHUNK + 1, CHUNK), jnp.int32),
            pltpu.VMEM((2, K, CHUNK, feat), jnp.float32),
            pltpu.VMEM_SHARED((N_PAD, feat), jnp.float32),
            pltpu.VMEM_SHARED((N_PAD, feat), jnp.float32),
            pltpu.SemaphoreType.DMA((2, K)),
            pltpu.SemaphoreType.DMA((2, K)),
        ],
    )


_agg32 = _make_agg(32)
_agg16 = _make_agg(16)


_ZPAD = N_PAD - N


def _tc_first_body(x_ref, deg_ref, w_ref, outa_ref, outb_ref):
    norm_s = _norm_col(deg_ref[...], 0)[:N]
    h = jnp.dot(x_ref[...], w_ref[...], preferred_element_type=jnp.float32)
    h = h * norm_s
    zp = jnp.zeros((_ZPAD, 32), jnp.float32)
    outa_ref[...] = jnp.concatenate([h[:, :32], zp], axis=0)
    outb_ref[...] = jnp.concatenate([h[:, 32:], zp], axis=0)


def _tc_mid2_body(pa_ref, pb_ref, deg_ref, b_ref, w_ref, out_ref):
    norm_d = _norm_col(deg_ref[...], 4)
    norm_s = _norm_col(deg_ref[...], 0)
    za = jnp.maximum((pa_ref[0] + pa_ref[1]) * norm_d + b_ref[:, :32], 0.0)
    zb = jnp.maximum((pb_ref[0] + pb_ref[1]) * norm_d + b_ref[:, 32:], 0.0)
    h = (jnp.dot(za, w_ref[:32], preferred_element_type=jnp.float32)
         + jnp.dot(zb, w_ref[32:], preferred_element_type=jnp.float32))
    out_ref[...] = h * norm_s


def _tc_mid_body(p_ref, deg_ref, b_ref, w_ref, out_ref):
    norm_d = _norm_col(deg_ref[...], 4)
    norm_s = _norm_col(deg_ref[...], 0)
    agg = p_ref[0] + p_ref[1]
    z = jnp.maximum(agg * norm_d + b_ref[...], 0.0)
    out_ref[...] = jnp.dot(z, w_ref[...],
                           preferred_element_type=jnp.float32) * norm_s


def _tc_out_body(p_ref, deg_ref, b_ref, out_ref):
    norm_d = _norm_col(deg_ref[...], 4)
    agg = p_ref[0] + p_ref[1]
    z = jnp.maximum(agg * norm_d + b_ref[...], 0.0)
    m = jnp.max(z, axis=1, keepdims=True)
    e = jnp.exp(z - m)
    out_ref[...] = e / jnp.sum(e, axis=1, keepdims=True)


def _tc_first(x, deg, w):
    return pl.pallas_call(
        _tc_first_body,
        out_shape=(jax.ShapeDtypeStruct((N_PAD, 32), jnp.float32),
                   jax.ShapeDtypeStruct((N_PAD, 32), jnp.float32)),
    )(x, deg, w)


def _tc_mid2(pa, pb, deg, b, w):
    return pl.pallas_call(
        _tc_mid2_body,
        out_shape=jax.ShapeDtypeStruct((N_PAD, w.shape[1]), jnp.float32),
    )(pa, pb, deg, b, w)


def _tc_mid(p, deg, b, w):
    return pl.pallas_call(
        _tc_mid_body,
        out_shape=jax.ShapeDtypeStruct((N_PAD, w.shape[1]), jnp.float32),
    )(p, deg, b, w)


def _tc_out(p, deg, b):
    return pl.pallas_call(
        _tc_out_body,
        out_shape=jax.ShapeDtypeStruct((N_PAD, b.shape[1]), jnp.float32),
    )(p, deg, b)


def kernel(x, edge_index, W1, b1, s1, W2, b2, s2, W3, b3, s3):
    ei = jnp.transpose(edge_index.reshape(2, NROWS, CHUNK), (1, 0, 2))

    eye = jnp.zeros((2, 1, 8), jnp.float32).at[0, 0, 0].set(1.0)
    eye = eye.at[1, 0, 4].set(1.0)
    ones2 = jnp.broadcast_to(eye, (2, CHUNK, 8))
    z8 = jnp.zeros((N_PAD, 8), jnp.float32)
    z16 = jnp.zeros((N_PAD, 16), jnp.float32)
    z32 = jnp.zeros((N_PAD, 32), jnp.float32)

    deg = _deg_kernel(ei, ones2, z8)

    h1a, h1b = _tc_first(x, deg, W1)
    p1a, p1b = _l1_kernel(h1a, h1b, ei, z32)
    h2 = _tc_mid2(p1a, p1b, deg, (b1 + s1).reshape(1, 64), W2)
    p2 = _agg32(h2, ei, z32)
    h3 = _tc_mid(p2, deg, (b2 + s2).reshape(1, 32), W3)
    p3 = _agg16(h3, ei, z16)
    out = _tc_out(p3, deg, (b3 + s3).reshape(1, 16))
    return out[:N]


# speedup vs baseline: 19.0864x; 1.0418x over previous
import jax
import jax.numpy as jnp
from jax import lax
from jax.experimental import pallas as pl
from jax.experimental.pallas import tpu as pltpu
from jax.experimental.pallas import tpu_sc as plsc

N = 10000
N_PAD = 10112
ROWS_PER_TILE = N_PAD // 16
E = 320000
NT = 32
CHUNK = 128
NROWS = E // CHUNK
NCHUNK = NROWS // NT
NEXTRA = NROWS - NCHUNK * NT
K = 6
NG = NCHUNK // K

_MESH = plsc.VectorSubcoreMesh(core_axis_name="c", subcore_axis_name="s")
_SC_PARAMS = pltpu.CompilerParams(use_tc_tiling_on_sc=False)


def _norm_col(deg_pair, col):
    deg = deg_pair[0] + deg_pair[1]
    nrm = jnp.where(deg > 0, lax.rsqrt(jnp.maximum(deg, 1.0)), 0.0)
    return nrm[:, col:col + 1]


def _load_indices(ei_hbm, idx_s, idx_d, wid):
    pltpu.sync_copy(ei_hbm.at[pl.ds(NCHUNK * wid, NCHUNK), 0],
                    idx_s.at[pl.ds(0, NCHUNK)])
    pltpu.sync_copy(ei_hbm.at[pl.ds(NCHUNK * wid, NCHUNK), 1],
                    idx_d.at[pl.ds(0, NCHUNK)])

    @pl.when(wid < NEXTRA)
    def _():
        pltpu.sync_copy(ei_hbm.at[NCHUNK * NT + wid, 0], idx_s.at[NCHUNK])
        pltpu.sync_copy(ei_hbm.at[NCHUNK * NT + wid, 1], idx_d.at[NCHUNK])


def _deg_body(ei_hbm, ones_hbm, zeros_hbm, out_hbm,
              idx_s, idx_d, ones_v, acc, dsem):
    c = lax.axis_index("c")
    s = lax.axis_index("s")
    wid = c * 16 + s
    r0 = s * ROWS_PER_TILE
    pltpu.sync_copy(zeros_hbm.at[pl.ds(r0, ROWS_PER_TILE)],
                    acc.at[pl.ds(r0, ROWS_PER_TILE)])
    _load_indices(ei_hbm, idx_s, idx_d, wid)
    pltpu.sync_copy(ones_hbm, ones_v)
    plsc.subcore_barrier()

    def body(j, carry):
        slot = lax.rem(j, 4)

        @pl.when(j >= 4)
        def _():
            pltpu.make_async_copy(ones_v.at[0], acc.at[idx_s.at[j - 4]],
                                  dsem.at[slot, 0]).wait()
            pltpu.make_async_copy(ones_v.at[1], acc.at[idx_d.at[j - 4]],
                                  dsem.at[slot, 1]).wait()

        pltpu.async_copy(ones_v.at[0], acc.at[idx_s.at[j]],
                         dsem.at[slot, 0], add=True)
        pltpu.async_copy(ones_v.at[1], acc.at[idx_d.at[j]],
                         dsem.at[slot, 1], add=True)
        return carry

    lax.fori_loop(0, NCHUNK, body, 0)

    def drain(j, carry):
        slot = lax.rem(j, 4)
        pltpu.make_async_copy(ones_v.at[0], acc.at[idx_s.at[j]],
                              dsem.at[slot, 0]).wait()
        pltpu.make_async_copy(ones_v.at[1], acc.at[idx_d.at[j]],
                              dsem.at[slot, 1]).wait()
        return carry

    lax.fori_loop(NCHUNK - 4, NCHUNK, drain, 0)

    @pl.when(wid < NEXTRA)
    def _():
        pltpu.sync_copy(ones_v.at[0], acc.at[idx_s.at[NCHUNK]], add=True)
        pltpu.sync_copy(ones_v.at[1], acc.at[idx_d.at[NCHUNK]], add=True)

    plsc.subcore_barrier()
    pltpu.sync_copy(acc.at[pl.ds(r0, ROWS_PER_TILE)],
                    out_hbm.at[c, pl.ds(r0, ROWS_PER_TILE)])


_deg_kernel = pl.kernel(
    _deg_body,
    out_type=jax.ShapeDtypeStruct((2, N_PAD, 8), jnp.float32),
    mesh=_MESH,
    compiler_params=_SC_PARAMS,
    scratch_types=[
        pltpu.VMEM((NCHUNK + 1, CHUNK), jnp.int32),
        pltpu.VMEM((NCHUNK + 1, CHUNK), jnp.int32),
        pltpu.VMEM((2, CHUNK, 8), jnp.float32),
        pltpu.VMEM_SHARED((N_PAD, 8), jnp.float32),
        pltpu.SemaphoreType.DMA((4, 2)),
    ],
)


def _agg_pass(h_sp, acc, idx_s, idx_d, rows, gsem, ssem, wid):
    def fire_gathers(g, bank):
        for k in range(K):
            pltpu.async_copy(h_sp.at[idx_s.at[g * K + k]],
                             rows.at[bank, k], gsem.at[bank, k])

    fire_gathers(0, 0)

    def body(g, carry):
        bank = lax.rem(g, 2)

        @pl.when(g < NG - 1)
        def _():
            fire_gathers(g + 1, 1 - bank)

        for k in range(K):
            pltpu.make_async_copy(h_sp.at[idx_s.at[g * K + k]],
                                  rows.at[bank, k],
                                  gsem.at[bank, k]).wait()
            pltpu.async_copy(rows.at[bank, k],
                             acc.at[idx_d.at[g * K + k]],
                             ssem.at[bank, k], add=True)
        for k in range(K):
            pltpu.make_async_copy(rows.at[bank, k],
                                  acc.at[idx_d.at[g * K + k]],
                                  ssem.at[bank, k]).wait()
        return carry

    lax.fori_loop(0, NG, body, 0)

    @pl.when(wid < NEXTRA)
    def _():
        pltpu.sync_copy(h_sp.at[idx_s.at[NCHUNK]], rows.at[0, 0])
        pltpu.sync_copy(rows.at[0, 0], acc.at[idx_d.at[NCHUNK]], add=True)


def _make_agg_body(feat):
    def _agg_body(h_hbm, ei_hbm, zeros_hbm, out_hbm,
                  idx_s, idx_d, rows, h_sp, acc, gsem, ssem):
        c = lax.axis_index("c")
        s = lax.axis_index("s")
        wid = c * 16 + s
        r0 = s * ROWS_PER_TILE
        pltpu.sync_copy(h_hbm.at[pl.ds(r0, ROWS_PER_TILE)],
                        h_sp.at[pl.ds(r0, ROWS_PER_TILE)])
        pltpu.sync_copy(zeros_hbm.at[pl.ds(r0, ROWS_PER_TILE)],
                        acc.at[pl.ds(r0, ROWS_PER_TILE)])
        _load_indices(ei_hbm, idx_s, idx_d, wid)
        plsc.subcore_barrier()
        _agg_pass(h_sp, acc, idx_s, idx_d, rows, gsem, ssem, wid)
        plsc.subcore_barrier()
        pltpu.sync_copy(acc.at[pl.ds(r0, ROWS_PER_TILE)],
                        out_hbm.at[c, pl.ds(r0, ROWS_PER_TILE)])

    return _agg_body


def _l1_body(ha_hbm, hb_hbm, ei_hbm, zeros_hbm, outa_hbm, outb_hbm,
             idx_s, idx_d, rows, h_spa, h_spb, acc, gsem, ssem):
    c = lax.axis_index("c")
    s = lax.axis_index("s")
    wid = c * 16 + s
    r0 = s * ROWS_PER_TILE
    pltpu.sync_copy(ha_hbm.at[pl.ds(r0, ROWS_PER_TILE)],
                    h_spa.at[pl.ds(r0, ROWS_PER_TILE)])
    pltpu.sync_copy(hb_hbm.at[pl.ds(r0, ROWS_PER_TILE)],
                    h_spb.at[pl.ds(r0, ROWS_PER_TILE)])
    pltpu.sync_copy(zeros_hbm.at[pl.ds(r0, ROWS_PER_TILE)],
                    acc.at[pl.ds(r0, ROWS_PER_TILE)])
    _load_indices(ei_hbm, idx_s, idx_d, wid)
    plsc.subcore_barrier()
    _agg_pass(h_spa, acc, idx_s, idx_d, rows, gsem, ssem, wid)
    plsc.subcore_barrier()
    pltpu.sync_copy(acc.at[pl.ds(r0, ROWS_PER_TILE)],
                    outa_hbm.at[c, pl.ds(r0, ROWS_PER_TILE)])
    pltpu.sync_copy(zeros_hbm.at[pl.ds(r0, ROWS_PER_TILE)],
                    acc.at[pl.ds(r0, ROWS_PER_TILE)])
    plsc.subcore_barrier()
    _agg_pass(h_spb, acc, idx_s, idx_d, rows, gsem, ssem, wid)
    plsc.subcore_barrier()
    pltpu.sync_copy(acc.at[pl.ds(r0, ROWS_PER_TILE)],
                    outb_hbm.at[c, pl.ds(r0, ROWS_PER_TILE)])


_l1_kernel = pl.kernel(
    _l1_body,
    out_type=(jax.ShapeDtypeStruct((2, N_PAD, 32), jnp.float32),
              jax.ShapeDtypeStruct((2, N_PAD, 32), jnp.float32)),
    mesh=_MESH,
    compiler_params=_SC_PARAMS,
    scratch_types=[
        pltpu.VMEM((NCHUNK + 1, CHUNK), jnp.int32),
        pltpu.VMEM((NCHUNK + 1, CHUNK), jnp.int32),
        pltpu.VMEM((2, K, CHUNK, 32), jnp.float32),
        pltpu.VMEM_SHARED((N_PAD, 32), jnp.float32),
        pltpu.VMEM_SHARED((N_PAD, 32), jnp.float32),
        pltpu.VMEM_SHARED((N_PAD, 32), jnp.float32),
        pltpu.SemaphoreType.DMA((2, K)),
        pltpu.SemaphoreType.DMA((2, K)),
    ],
)


def _make_agg(feat):
    return pl.kernel(
        _make_agg_body(feat),
        out_type=jax.ShapeDtypeStruct((2, N_PAD, feat), jnp.float32),
        mesh=_MESH,
        compiler_params=_SC_PARAMS,
        scratch_types=[
            pltpu.VMEM((NCHUNK + 1, CHUNK), jnp.int32),
            pltpu.VMEM((NCHUNK + 1, CHUNK), jnp.int32),
            pltpu.VMEM((2, K, CHUNK, feat), jnp.float32),
            pltpu.VMEM_SHARED((N_PAD, feat), jnp.float32),
            pltpu.VMEM_SHARED((N_PAD, feat), jnp.float32),
            pltpu.SemaphoreType.DMA((2, K)),
            pltpu.SemaphoreType.DMA((2, K)),
        ],
    )


_agg32 = _make_agg(32)
_agg16 = _make_agg(16)


_ZPAD = N_PAD - N


def _tc_first_body(x_ref, deg_ref, w_ref, outa_ref, outb_ref):
    norm_s = _norm_col(deg_ref[...], 0)[:N]
    h = jnp.dot(x_ref[...], w_ref[...], preferred_element_type=jnp.float32)
    h = h * norm_s
    zp = jnp.zeros((_ZPAD, 32), jnp.float32)
    outa_ref[...] = jnp.concatenate([h[:, :32], zp], axis=0)
    outb_ref[...] = jnp.concatenate([h[:, 32:], zp], axis=0)


def _tc_mid2_body(pa_ref, pb_ref, deg_ref, b_ref, w_ref, out_ref):
    norm_d = _norm_col(deg_ref[...], 4)
    norm_s = _norm_col(deg_ref[...], 0)
    za = jnp.maximum((pa_ref[0] + pa_ref[1]) * norm_d + b_ref[:, :32], 0.0)
    zb = jnp.maximum((pb_ref[0] + pb_ref[1]) * norm_d + b_ref[:, 32:], 0.0)
    h = (jnp.dot(za, w_ref[:32], preferred_element_type=jnp.float32)
         + jnp.dot(zb, w_ref[32:], preferred_element_type=jnp.float32))
    out_ref[...] = h * norm_s


def _tc_mid_body(p_ref, deg_ref, b_ref, w_ref, out_ref):
    norm_d = _norm_col(deg_ref[...], 4)
    norm_s = _norm_col(deg_ref[...], 0)
    agg = p_ref[0] + p_ref[1]
    z = jnp.maximum(agg * norm_d + b_ref[...], 0.0)
    out_ref[...] = jnp.dot(z, w_ref[...],
                           preferred_element_type=jnp.float32) * norm_s


def _tc_out_body(p_ref, deg_ref, b_ref, out_ref):
    norm_d = _norm_col(deg_ref[...], 4)[:N]
    agg = (p_ref[0] + p_ref[1])[:N]
    z = jnp.maximum(agg * norm_d + b_ref[...], 0.0)
    m = jnp.max(z, axis=1, keepdims=True)
    e = jnp.exp(z - m)
    out_ref[...] = e / jnp.sum(e, axis=1, keepdims=True)


def _tc_first(x, deg, w):
    return pl.pallas_call(
        _tc_first_body,
        out_shape=(jax.ShapeDtypeStruct((N_PAD, 32), jnp.float32),
                   jax.ShapeDtypeStruct((N_PAD, 32), jnp.float32)),
    )(x, deg, w)


_GB = 8
_BR = N_PAD // _GB


def _pspec(feat):
    return pl.BlockSpec((2, _BR, feat), lambda i: (0, i, 0))


def _fullspec(shape):
    nd = len(shape)
    return pl.BlockSpec(shape, lambda i: (0,) * nd)


def _tc_mid2(pa, pb, deg, b, w):
    return pl.pallas_call(
        _tc_mid2_body,
        grid=(_GB,),
        in_specs=[_pspec(32), _pspec(32), _pspec(8),
                  _fullspec(b.shape), _fullspec(w.shape)],
        out_specs=pl.BlockSpec((_BR, w.shape[1]), lambda i: (i, 0)),
        out_shape=jax.ShapeDtypeStruct((N_PAD, w.shape[1]), jnp.float32),
    )(pa, pb, deg, b, w)


def _tc_mid(p, deg, b, w):
    return pl.pallas_call(
        _tc_mid_body,
        grid=(_GB,),
        in_specs=[_pspec(p.shape[2]), _pspec(8),
                  _fullspec(b.shape), _fullspec(w.shape)],
        out_specs=pl.BlockSpec((_BR, w.shape[1]), lambda i: (i, 0)),
        out_shape=jax.ShapeDtypeStruct((N_PAD, w.shape[1]), jnp.float32),
    )(p, deg, b, w)


def _tc_out(p, deg, b):
    return pl.pallas_call(
        _tc_out_body,
        out_shape=jax.ShapeDtypeStruct((N, b.shape[1]), jnp.float32),
    )(p, deg, b)


def kernel(x, edge_index, W1, b1, s1, W2, b2, s2, W3, b3, s3):
    ei = jnp.transpose(edge_index.reshape(2, NROWS, CHUNK), (1, 0, 2))

    eye = jnp.zeros((2, 1, 8), jnp.float32).at[0, 0, 0].set(1.0)
    eye = eye.at[1, 0, 4].set(1.0)
    ones2 = jnp.broadcast_to(eye, (2, CHUNK, 8))
    z8 = jnp.zeros((N_PAD, 8), jnp.float32)
    z16 = jnp.zeros((N_PAD, 16), jnp.float32)
    z32 = jnp.zeros((N_PAD, 32), jnp.float32)

    deg = _deg_kernel(ei, ones2, z8)

    h1a, h1b = _tc_first(x, deg, W1)
    p1a, p1b = _l1_kernel(h1a, h1b, ei, z32)
    h2 = _tc_mid2(p1a, p1b, deg, (b1 + s1).reshape(1, 64), W2)
    p2 = _agg32(h2, ei, z32)
    h3 = _tc_mid(p2, deg, (b2 + s2).reshape(1, 32), W3)
    p3 = _agg16(h3, ei, z16)
    return _tc_out(p3, deg, (b3 + s3).reshape(1, 16))


# speedup vs baseline: 19.7658x; 1.0356x over previous
import jax
import jax.numpy as jnp
from jax import lax
from jax.experimental import pallas as pl
from jax.experimental.pallas import tpu as pltpu
from jax.experimental.pallas import tpu_sc as plsc

N = 10000
N_PAD = 10112
ROWS_PER_TILE = N_PAD // 16
E = 320000
NT = 32
CHUNK = 128
NROWS = E // CHUNK
NCHUNK = NROWS // NT
NEXTRA = NROWS - NCHUNK * NT
K = 3
NG = NCHUNK // K

_MESH = plsc.VectorSubcoreMesh(core_axis_name="c", subcore_axis_name="s")
_SC_PARAMS = pltpu.CompilerParams(use_tc_tiling_on_sc=False)


def _norm_col(deg_pair, col):
    deg = deg_pair[0] + deg_pair[1]
    nrm = jnp.where(deg > 0, lax.rsqrt(jnp.maximum(deg, 1.0)), 0.0)
    return nrm[:, col:col + 1]


def _load_indices(ei_hbm, idx_s, idx_d, wid):
    pltpu.sync_copy(ei_hbm.at[pl.ds(NCHUNK * wid, NCHUNK), 0],
                    idx_s.at[pl.ds(0, NCHUNK)])
    pltpu.sync_copy(ei_hbm.at[pl.ds(NCHUNK * wid, NCHUNK), 1],
                    idx_d.at[pl.ds(0, NCHUNK)])

    @pl.when(wid < NEXTRA)
    def _():
        pltpu.sync_copy(ei_hbm.at[NCHUNK * NT + wid, 0], idx_s.at[NCHUNK])
        pltpu.sync_copy(ei_hbm.at[NCHUNK * NT + wid, 1], idx_d.at[NCHUNK])


def _deg_body(ei_hbm, ones_hbm, zeros_hbm, out_hbm,
              idx_s, idx_d, ones_v, acc, dsem):
    c = lax.axis_index("c")
    s = lax.axis_index("s")
    wid = c * 16 + s
    r0 = s * ROWS_PER_TILE
    pltpu.sync_copy(zeros_hbm.at[pl.ds(r0, ROWS_PER_TILE)],
                    acc.at[pl.ds(r0, ROWS_PER_TILE)])
    _load_indices(ei_hbm, idx_s, idx_d, wid)
    pltpu.sync_copy(ones_hbm, ones_v)
    plsc.subcore_barrier()

    def body(j, carry):
        slot = lax.rem(j, 4)

        @pl.when(j >= 4)
        def _():
            pltpu.make_async_copy(ones_v.at[0], acc.at[idx_s.at[j - 4]],
                                  dsem.at[slot, 0]).wait()
            pltpu.make_async_copy(ones_v.at[1], acc.at[idx_d.at[j - 4]],
                                  dsem.at[slot, 1]).wait()

        pltpu.async_copy(ones_v.at[0], acc.at[idx_s.at[j]],
                         dsem.at[slot, 0], add=True)
        pltpu.async_copy(ones_v.at[1], acc.at[idx_d.at[j]],
                         dsem.at[slot, 1], add=True)
        return carry

    lax.fori_loop(0, NCHUNK, body, 0)

    def drain(j, carry):
        slot = lax.rem(j, 4)
        pltpu.make_async_copy(ones_v.at[0], acc.at[idx_s.at[j]],
                              dsem.at[slot, 0]).wait()
        pltpu.make_async_copy(ones_v.at[1], acc.at[idx_d.at[j]],
                              dsem.at[slot, 1]).wait()
        return carry

    lax.fori_loop(NCHUNK - 4, NCHUNK, drain, 0)

    @pl.when(wid < NEXTRA)
    def _():
        pltpu.sync_copy(ones_v.at[0], acc.at[idx_s.at[NCHUNK]], add=True)
        pltpu.sync_copy(ones_v.at[1], acc.at[idx_d.at[NCHUNK]], add=True)

    plsc.subcore_barrier()
    pltpu.sync_copy(acc.at[pl.ds(r0, ROWS_PER_TILE)],
                    out_hbm.at[c, pl.ds(r0, ROWS_PER_TILE)])


_deg_kernel = pl.kernel(
    _deg_body,
    out_type=jax.ShapeDtypeStruct((2, N_PAD, 8), jnp.float32),
    mesh=_MESH,
    compiler_params=_SC_PARAMS,
    scratch_types=[
        pltpu.VMEM((NCHUNK + 1, CHUNK), jnp.int32),
        pltpu.VMEM((NCHUNK + 1, CHUNK), jnp.int32),
        pltpu.VMEM((2, CHUNK, 8), jnp.float32),
        pltpu.VMEM_SHARED((N_PAD, 8), jnp.float32),
        pltpu.SemaphoreType.DMA((4, 2)),
    ],
)


def _agg_pass(h_sp, acc, idx_s, idx_d, rows, gsem, ssem, wid):
    def fire_gathers(g, bank):
        for k in range(K):
            pltpu.async_copy(h_sp.at[idx_s.at[g * K + k]],
                             rows.at[bank, k], gsem.at[bank, k])

    def drain_scatters(g, bank):
        for k in range(K):
            pltpu.make_async_copy(rows.at[bank, k],
                                  acc.at[idx_d.at[g * K + k]],
                                  ssem.at[bank, k]).wait()

    fire_gathers(0, 0)

    def body(g, carry):
        bank = lax.rem(g, 3)
        nb = lax.rem(g + 1, 3)

        @pl.when(g < NG - 1)
        def _():
            @pl.when(g >= 2)
            def _():
                drain_scatters(g - 2, nb)

            fire_gathers(g + 1, nb)

        for k in range(K):
            pltpu.make_async_copy(h_sp.at[idx_s.at[g * K + k]],
                                  rows.at[bank, k],
                                  gsem.at[bank, k]).wait()
            pltpu.async_copy(rows.at[bank, k],
                             acc.at[idx_d.at[g * K + k]],
                             ssem.at[bank, k], add=True)
        return carry

    lax.fori_loop(0, NG, body, 0)
    drain_scatters(NG - 2, (NG - 2) % 3)
    drain_scatters(NG - 1, (NG - 1) % 3)

    @pl.when(wid < NEXTRA)
    def _():
        pltpu.sync_copy(h_sp.at[idx_s.at[NCHUNK]], rows.at[0, 0])
        pltpu.sync_copy(rows.at[0, 0], acc.at[idx_d.at[NCHUNK]], add=True)


def _make_agg_body(feat):
    def _agg_body(h_hbm, ei_hbm, zeros_hbm, out_hbm,
                  idx_s, idx_d, rows, h_sp, acc, gsem, ssem):
        c = lax.axis_index("c")
        s = lax.axis_index("s")
        wid = c * 16 + s
        r0 = s * ROWS_PER_TILE
        pltpu.sync_copy(h_hbm.at[pl.ds(r0, ROWS_PER_TILE)],
                        h_sp.at[pl.ds(r0, ROWS_PER_TILE)])
        pltpu.sync_copy(zeros_hbm.at[pl.ds(r0, ROWS_PER_TILE)],
                        acc.at[pl.ds(r0, ROWS_PER_TILE)])
        _load_indices(ei_hbm, idx_s, idx_d, wid)
        plsc.subcore_barrier()
        _agg_pass(h_sp, acc, idx_s, idx_d, rows, gsem, ssem, wid)
        plsc.subcore_barrier()
        pltpu.sync_copy(acc.at[pl.ds(r0, ROWS_PER_TILE)],
                        out_hbm.at[c, pl.ds(r0, ROWS_PER_TILE)])

    return _agg_body


def _l1_body(ha_hbm, hb_hbm, ei_hbm, zeros_hbm, outa_hbm, outb_hbm,
             idx_s, idx_d, rows, h_spa, h_spb, acc, gsem, ssem):
    c = lax.axis_index("c")
    s = lax.axis_index("s")
    wid = c * 16 + s
    r0 = s * ROWS_PER_TILE
    pltpu.sync_copy(ha_hbm.at[pl.ds(r0, ROWS_PER_TILE)],
                    h_spa.at[pl.ds(r0, ROWS_PER_TILE)])
    pltpu.sync_copy(hb_hbm.at[pl.ds(r0, ROWS_PER_TILE)],
                    h_spb.at[pl.ds(r0, ROWS_PER_TILE)])
    pltpu.sync_copy(zeros_hbm.at[pl.ds(r0, ROWS_PER_TILE)],
                    acc.at[pl.ds(r0, ROWS_PER_TILE)])
    _load_indices(ei_hbm, idx_s, idx_d, wid)
    plsc.subcore_barrier()
    _agg_pass(h_spa, acc, idx_s, idx_d, rows, gsem, ssem, wid)
    plsc.subcore_barrier()
    pltpu.sync_copy(acc.at[pl.ds(r0, ROWS_PER_TILE)],
                    outa_hbm.at[c, pl.ds(r0, ROWS_PER_TILE)])
    pltpu.sync_copy(zeros_hbm.at[pl.ds(r0, ROWS_PER_TILE)],
                    acc.at[pl.ds(r0, ROWS_PER_TILE)])
    plsc.subcore_barrier()
    _agg_pass(h_spb, acc, idx_s, idx_d, rows, gsem, ssem, wid)
    plsc.subcore_barrier()
    pltpu.sync_copy(acc.at[pl.ds(r0, ROWS_PER_TILE)],
                    outb_hbm.at[c, pl.ds(r0, ROWS_PER_TILE)])


_l1_kernel = pl.kernel(
    _l1_body,
    out_type=(jax.ShapeDtypeStruct((2, N_PAD, 32), jnp.float32),
              jax.ShapeDtypeStruct((2, N_PAD, 32), jnp.float32)),
    mesh=_MESH,
    compiler_params=_SC_PARAMS,
    scratch_types=[
        pltpu.VMEM((NCHUNK + 1, CHUNK), jnp.int32),
        pltpu.VMEM((NCHUNK + 1, CHUNK), jnp.int32),
        pltpu.VMEM((3, K, CHUNK, 32), jnp.float32),
        pltpu.VMEM_SHARED((N_PAD, 32), jnp.float32),
        pltpu.VMEM_SHARED((N_PAD, 32), jnp.float32),
        pltpu.VMEM_SHARED((N_PAD, 32), jnp.float32),
        pltpu.SemaphoreType.DMA((3, K)),
        pltpu.SemaphoreType.DMA((3, K)),
    ],
)


def _make_agg(feat):
    return pl.kernel(
        _make_agg_body(feat),
        out_type=jax.ShapeDtypeStruct((2, N_PAD, feat), jnp.float32),
        mesh=_MESH,
        compiler_params=_SC_PARAMS,
        scratch_types=[
            pltpu.VMEM((NCHUNK + 1, CHUNK), jnp.int32),
            pltpu.VMEM((NCHUNK + 1, CHUNK), jnp.int32),
            pltpu.VMEM((3, K, CHUNK, feat), jnp.float32),
            pltpu.VMEM_SHARED((N_PAD, feat), jnp.float32),
            pltpu.VMEM_SHARED((N_PAD, feat), jnp.float32),
            pltpu.SemaphoreType.DMA((3, K)),
            pltpu.SemaphoreType.DMA((3, K)),
        ],
    )


_agg32 = _make_agg(32)
_agg16 = _make_agg(16)


_ZPAD = N_PAD - N


def _tc_first_body(x_ref, deg_ref, w_ref, outa_ref, outb_ref):
    norm_s = _norm_col(deg_ref[...], 0)[:N]
    h = jnp.dot(x_ref[...], w_ref[...], preferred_element_type=jnp.float32)
    h = h * norm_s
    zp = jnp.zeros((_ZPAD, 32), jnp.float32)
    outa_ref[...] = jnp.concatenate([h[:, :32], zp], axis=0)
    outb_ref[...] = jnp.concatenate([h[:, 32:], zp], axis=0)


def _tc_mid2_body(pa_ref, pb_ref, deg_ref, b_ref, w_ref, out_ref):
    norm_d = _norm_col(deg_ref[...], 4)
    norm_s = _norm_col(deg_ref[...], 0)
    za = jnp.maximum((pa_ref[0] + pa_ref[1]) * norm_d + b_ref[:, :32], 0.0)
    zb = jnp.maximum((pb_ref[0] + pb_ref[1]) * norm_d + b_ref[:, 32:], 0.0)
    h = (jnp.dot(za, w_ref[:32], preferred_element_type=jnp.float32)
         + jnp.dot(zb, w_ref[32:], preferred_element_type=jnp.float32))
    out_ref[...] = h * norm_s


def _tc_mid_body(p_ref, deg_ref, b_ref, w_ref, out_ref):
    norm_d = _norm_col(deg_ref[...], 4)
    norm_s = _norm_col(deg_ref[...], 0)
    agg = p_ref[0] + p_ref[1]
    z = jnp.maximum(agg * norm_d + b_ref[...], 0.0)
    out_ref[...] = jnp.dot(z, w_ref[...],
                           preferred_element_type=jnp.float32) * norm_s


def _tc_out_body(p_ref, deg_ref, b_ref, out_ref):
    norm_d = _norm_col(deg_ref[...], 4)[:N]
    agg = (p_ref[0] + p_ref[1])[:N]
    z = jnp.maximum(agg * norm_d + b_ref[...], 0.0)
    m = jnp.max(z, axis=1, keepdims=True)
    e = jnp.exp(z - m)
    out_ref[...] = e / jnp.sum(e, axis=1, keepdims=True)


def _tc_first(x, deg, w):
    return pl.pallas_call(
        _tc_first_body,
        out_shape=(jax.ShapeDtypeStruct((N_PAD, 32), jnp.float32),
                   jax.ShapeDtypeStruct((N_PAD, 32), jnp.float32)),
    )(x, deg, w)


_GB = 8
_BR = N_PAD // _GB


def _pspec(feat):
    return pl.BlockSpec((2, _BR, feat), lambda i: (0, i, 0))


def _fullspec(shape):
    nd = len(shape)
    return pl.BlockSpec(shape, lambda i: (0,) * nd)


def _tc_mid2(pa, pb, deg, b, w):
    return pl.pallas_call(
        _tc_mid2_body,
        grid=(_GB,),
        in_specs=[_pspec(32), _pspec(32), _pspec(8),
                  _fullspec(b.shape), _fullspec(w.shape)],
        out_specs=pl.BlockSpec((_BR, w.shape[1]), lambda i: (i, 0)),
        out_shape=jax.ShapeDtypeStruct((N_PAD, w.shape[1]), jnp.float32),
    )(pa, pb, deg, b, w)


def _tc_mid(p, deg, b, w):
    return pl.pallas_call(
        _tc_mid_body,
        grid=(_GB,),
        in_specs=[_pspec(p.shape[2]), _pspec(8),
                  _fullspec(b.shape), _fullspec(w.shape)],
        out_specs=pl.BlockSpec((_BR, w.shape[1]), lambda i: (i, 0)),
        out_shape=jax.ShapeDtypeStruct((N_PAD, w.shape[1]), jnp.float32),
    )(p, deg, b, w)


def _tc_out(p, deg, b):
    return pl.pallas_call(
        _tc_out_body,
        out_shape=jax.ShapeDtypeStruct((N, b.shape[1]), jnp.float32),
    )(p, deg, b)


def kernel(x, edge_index, W1, b1, s1, W2, b2, s2, W3, b3, s3):
    ei = jnp.transpose(edge_index.reshape(2, NROWS, CHUNK), (1, 0, 2))

    eye = jnp.zeros((2, 1, 8), jnp.float32).at[0, 0, 0].set(1.0)
    eye = eye.at[1, 0, 4].set(1.0)
    ones2 = jnp.broadcast_to(eye, (2, CHUNK, 8))
    z8 = jnp.zeros((N_PAD, 8), jnp.float32)
    z16 = jnp.zeros((N_PAD, 16), jnp.float32)
    z32 = jnp.zeros((N_PAD, 32), jnp.float32)

    deg = _deg_kernel(ei, ones2, z8)

    h1a, h1b = _tc_first(x, deg, W1)
    p1a, p1b = _l1_kernel(h1a, h1b, ei, z32)
    h2 = _tc_mid2(p1a, p1b, deg, (b1 + s1).reshape(1, 64), W2)
    p2 = _agg32(h2, ei, z32)
    h3 = _tc_mid(p2, deg, (b2 + s2).reshape(1, 32), W3)
    p3 = _agg16(h3, ei, z16)
    return _tc_out(p3, deg, (b3 + s3).reshape(1, 16))
